# baseline (speedup 1.0000x reference)
import functools

import jax
import jax.numpy as jnp
from jax import lax
from jax.experimental import pallas as pl
from jax.experimental.pallas import tpu as pltpu
from jax.experimental.pallas import tpu_sc as plsc

N = 10000
E = 320000
R = 8
NP_ = 10240
NR = NP_ * R
NTILES = 32
EP = 327680
BT = 128
NB = EP // (NTILES * BT)
ROWS_PT = NR // 16
ZB = 128

_mesh = plsc.VectorSubcoreMesh(core_axis_name="c", subcore_axis_name="s")
_sc_params = pltpu.CompilerParams(use_tc_tiling_on_sc=False)


NBUF = 8
PF = 4


def _edge_sweep(tbl_hbm, gidx_v, sidx_v, rows_v, gsem, ssem, acc_s,
                cnt_fire=None):
    def g_start(j, b):
        pltpu.async_copy(tbl_hbm.at[gidx_v.at[j]], rows_v.at[b], gsem.at[b])

    def g_wait(b):
        pltpu.make_async_copy(tbl_hbm.at[gidx_v.at[0]], rows_v.at[b],
                              gsem.at[b]).wait()

    def s_start(j, b):
        pltpu.async_copy(rows_v.at[b], acc_s.at[sidx_v.at[j]], ssem.at[b],
                         add=True)

    def s_wait(b):
        pltpu.make_async_copy(rows_v.at[b], acc_s.at[sidx_v.at[0]],
                              ssem.at[b]).wait()

    def step(j, b, prefetch, wait_prev_scatter):
        g_wait(b)
        s_start(j, b)
        if cnt_fire is not None:
            cnt_fire(j)
        if prefetch:
            bp = (b + PF) % NBUF
            if wait_prev_scatter:
                s_wait(bp)
            g_start(j + PF, bp)

    for b in range(PF):
        g_start(b, b)
    for j in range(NBUF):
        step(j, j, prefetch=True, wait_prev_scatter=(j + PF >= NBUF))

    @pl.loop(NBUF, NB - NBUF, step=NBUF)
    def _(j0):
        for b in range(NBUF):
            step(j0 + b, b, prefetch=True, wait_prev_scatter=True)

    for j in range(NB - NBUF, NB):
        step(j, j % NBUF, prefetch=(j + PF < NB), wait_prev_scatter=True)
    for b in range(NBUF):
        s_wait(b)


def _sc_pass1_body(gidx_hbm, sidx_hbm, tbl_hbm, acc_out, cnt_out,
                   gidx_v, sidx_v, rows_v, ones_v, zbuf, zcnt, acc_s, cnt_s,
                   gsem, ssem, csem):
    cid = lax.axis_index("c")
    sid = lax.axis_index("s")
    tid = cid * 16 + sid

    @pl.loop(0, ZB)
    def _(i):
        zbuf[i] = jnp.zeros((16,), jnp.float32)

    @pl.loop(0, ZB, step=16)
    def _(i):
        zcnt[pl.ds(i, 16)] = jnp.zeros((16,), jnp.float32)

    for i in range(BT // 16):
        ones_v[pl.ds(i * 16, 16)] = jnp.ones((16,), jnp.float32)

    row0 = sid * ROWS_PT
    for k in range(ROWS_PT // ZB):
        pltpu.sync_copy(zbuf, acc_s.at[pl.ds(row0 + k * ZB, ZB)])
        pltpu.sync_copy(zcnt, cnt_s.at[pl.ds(row0 + k * ZB, ZB)])
    pltpu.sync_copy(gidx_hbm.at[pl.ds(tid * NB, NB)], gidx_v)
    pltpu.sync_copy(sidx_hbm.at[pl.ds(tid * NB, NB)], sidx_v)
    plsc.subcore_barrier()

    def cnt_fire(j):
        pltpu.async_copy(ones_v, cnt_s.at[sidx_v.at[j]], csem, add=True)

    _edge_sweep(tbl_hbm, gidx_v, sidx_v, rows_v, gsem, ssem, acc_s, cnt_fire)

    @pl.loop(0, NB)
    def _(_j):
        pltpu.make_async_copy(ones_v, cnt_s.at[sidx_v.at[0]], csem).wait()

    plsc.subcore_barrier()
    pltpu.sync_copy(acc_s.at[pl.ds(row0, ROWS_PT)],
                    acc_out.at[cid, pl.ds(row0, ROWS_PT)])
    pltpu.sync_copy(cnt_s.at[pl.ds(row0, ROWS_PT)],
                    cnt_out.at[cid, pl.ds(row0, ROWS_PT)])


_sc_pass1 = functools.partial(
    pl.kernel,
    out_type=[jax.ShapeDtypeStruct((2, NR, 16), jnp.float32),
              jax.ShapeDtypeStruct((2, NR), jnp.float32)],
    mesh=_mesh,
    scratch_types=[
        pltpu.VMEM((NB, BT), jnp.int32),
        pltpu.VMEM((NB, BT), jnp.int32),
        pltpu.VMEM((NBUF, BT, 16), jnp.float32),
        pltpu.VMEM((BT,), jnp.float32),
        pltpu.VMEM((ZB, 16), jnp.float32),
        pltpu.VMEM((ZB,), jnp.float32),
        pltpu.VMEM_SHARED((NR, 16), jnp.float32),
        pltpu.VMEM_SHARED((NR,), jnp.float32),
        pltpu.SemaphoreType.DMA((NBUF,)),
        pltpu.SemaphoreType.DMA((NBUF,)),
        pltpu.SemaphoreType.DMA,
    ],
    compiler_params=_sc_params,
)(_sc_pass1_body)


def _sc_pass2_body(gidx_hbm, sidx_hbm, tbl_hbm, acc_out,
                   gidx_v, sidx_v, rows_v, zbuf, acc_s, gsem, ssem):
    cid = lax.axis_index("c")
    sid = lax.axis_index("s")
    tid = cid * 16 + sid

    @pl.loop(0, ZB)
    def _(i):
        zbuf[i] = jnp.zeros((16,), jnp.float32)

    row0 = sid * ROWS_PT
    for k in range(ROWS_PT // ZB):
        pltpu.sync_copy(zbuf, acc_s.at[pl.ds(row0 + k * ZB, ZB)])
    pltpu.sync_copy(gidx_hbm.at[pl.ds(tid * NB, NB)], gidx_v)
    pltpu.sync_copy(sidx_hbm.at[pl.ds(tid * NB, NB)], sidx_v)
    plsc.subcore_barrier()

    _edge_sweep(tbl_hbm, gidx_v, sidx_v, rows_v, gsem, ssem, acc_s)

    plsc.subcore_barrier()
    pltpu.sync_copy(acc_s.at[pl.ds(row0, ROWS_PT)],
                    acc_out.at[cid, pl.ds(row0, ROWS_PT)])


_sc_pass2 = functools.partial(
    pl.kernel,
    out_type=jax.ShapeDtypeStruct((2, NR, 16), jnp.float32),
    mesh=_mesh,
    scratch_types=[
        pltpu.VMEM((NB, BT), jnp.int32),
        pltpu.VMEM((NB, BT), jnp.int32),
        pltpu.VMEM((NBUF, BT, 16), jnp.float32),
        pltpu.VMEM((ZB, 16), jnp.float32),
        pltpu.VMEM_SHARED((NR, 16), jnp.float32),
        pltpu.SemaphoreType.DMA((NBUF,)),
        pltpu.SemaphoreType.DMA((NBUF,)),
    ],
    compiler_params=_sc_params,
)(_sc_pass2_body)


EB_ROWS = E // BT
PAD_ROWS = EP // BT - EB_ROWS


def _idx_body(src_ref, dst_ref, et_ref, g_ref, s_ref, sp_ref):
    et = et_ref[...]
    g_ref[0:EB_ROWS] = src_ref[...] * R + et
    s_ref[0:EB_ROWS] = dst_ref[...] * R + et
    sp_ref[0:EB_ROWS] = src_ref[...]
    pidx = (lax.broadcasted_iota(jnp.int32, (PAD_ROWS, BT), 0) * BT
            + lax.broadcasted_iota(jnp.int32, (PAD_ROWS, BT), 1))
    spread = pidx % (NR - N * R)
    g_ref[EB_ROWS:] = spread
    s_ref[EB_ROWS:] = N * R + spread
    sp_ref[EB_ROWS:] = spread


_idx_call = pl.pallas_call(
    _idx_body,
    out_shape=[jax.ShapeDtypeStruct((EP // BT, BT), jnp.int32),
               jax.ShapeDtypeStruct((EP // BT, BT), jnp.int32),
               jax.ShapeDtypeStruct((EP // BT, BT), jnp.int32)],
)

BN = 1000


def _prep_body(x_ref, w_ref, r_ref, b_ref, xp_ref, o1_ref):
    xb = x_ref[...]
    xp_ref[...] = jnp.dot(xb, w_ref[...], preferred_element_type=jnp.float32)
    o1_ref[...] = jnp.dot(xb, r_ref[...],
                          preferred_element_type=jnp.float32) + b_ref[...]


_prep_call = pl.pallas_call(
    _prep_body,
    grid=(N // BN,),
    in_specs=[pl.BlockSpec((BN, 128), lambda i: (i, 0)),
              pl.BlockSpec((128, 128), lambda i: (0, 0)),
              pl.BlockSpec((128, 16), lambda i: (0, 0)),
              pl.BlockSpec((1, 16), lambda i: (0, 0))],
    out_specs=[pl.BlockSpec((BN, 128), lambda i: (i, 0)),
               pl.BlockSpec((BN, 16), lambda i: (i, 0))],
    out_shape=[jax.ShapeDtypeStruct((N, 128), jnp.float32),
               jax.ShapeDtypeStruct((N, 16), jnp.float32)],
)


def _mid_body(acc_ref, cnt_ref, o1_ref, h_ref):
    acc = (acc_ref[0] + acc_ref[1]).reshape(BN, R, 16)
    cnt = cnt_ref[0] + cnt_ref[1]
    inv = 1.0 / jnp.maximum(cnt, 1.0)
    ms = jnp.sum(acc * inv[:, :, None], axis=1)
    h_ref[...] = jnp.maximum(o1_ref[...] + ms, 0.0)


_mid_call = pl.pallas_call(
    _mid_body,
    grid=(N // BN,),
    in_specs=[pl.BlockSpec((2, BN * R, 16), lambda i: (0, i, 0)),
              pl.BlockSpec((2, BN, 8), lambda i: (0, i, 0)),
              pl.BlockSpec((BN, 16), lambda i: (i, 0))],
    out_specs=pl.BlockSpec((BN, 16), lambda i: (i, 0)),
    out_shape=jax.ShapeDtypeStruct((N, 16), jnp.float32),
)


def _fin_body(acc_ref, cnt_ref, h_ref, r2_ref, b2_ref, w_ref, out_ref):
    acc = (acc_ref[0] + acc_ref[1]).reshape(BN, R, 16)
    cnt = cnt_ref[0] + cnt_ref[1]
    inv = 1.0 / jnp.maximum(cnt, 1.0)
    m = acc * inv[:, :, None]
    s = jnp.dot(h_ref[...], r2_ref[...],
                preferred_element_type=jnp.float32) + b2_ref[...]
    for r in range(R):
        s = s + jnp.dot(m[:, r, :], w_ref[r],
                        preferred_element_type=jnp.float32)
    out_ref[...] = s


_fin_call = pl.pallas_call(
    _fin_body,
    grid=(N // BN,),
    in_specs=[pl.BlockSpec((2, BN * R, 16), lambda i: (0, i, 0)),
              pl.BlockSpec((2, BN, 8), lambda i: (0, i, 0)),
              pl.BlockSpec((BN, 16), lambda i: (i, 0)),
              pl.BlockSpec((16, 128), lambda i: (0, 0)),
              pl.BlockSpec((1, 128), lambda i: (0, 0)),
              pl.BlockSpec((8, 16, 128), lambda i: (0, 0, 0))],
    out_specs=pl.BlockSpec((BN, 128), lambda i: (i, 0)),
    out_shape=jax.ShapeDtypeStruct((N, 128), jnp.float32),
)


def kernel(x, edge_index, edge_type, W1, root1, b1, W2, root2, b2):
    src2 = edge_index[0].reshape(EB_ROWS, BT)
    dst2 = edge_index[1].reshape(EB_ROWS, BT)
    et2 = edge_type.reshape(EB_ROWS, BT)
    gidx2, sidx2, srcp2 = _idx_call(src2, dst2, et2)

    W1cat = W1.transpose(1, 0, 2).reshape(128, 128)
    xproj, o1root = _prep_call(x, W1cat, root1, b1.reshape(1, 16))

    acc1, cnt = _sc_pass1(gidx2, sidx2, xproj.reshape(N * R, 16))

    cntr = cnt.reshape(2, NP_, R)
    h = _mid_call(acc1, cntr, o1root)

    acc2 = _sc_pass2(srcp2, sidx2, h)
    out = _fin_call(acc2, cntr, h, root2, b2.reshape(1, 128), W2)
    return out


# speedup vs baseline: 1.8313x; 1.8313x over previous
import functools

import jax
import jax.numpy as jnp
from jax import lax
from jax.experimental import pallas as pl
from jax.experimental.pallas import tpu as pltpu
from jax.experimental.pallas import tpu_sc as plsc

N = 10000
E = 320000
R = 8
NP_ = 10240
NR = NP_ * R
NTILES = 32
EP = 327680
BT = 128
NB = EP // (NTILES * BT)
ROWS_PT = NR // 16
ZB = 128

_mesh = plsc.VectorSubcoreMesh(core_axis_name="c", subcore_axis_name="s")
_sc_params = pltpu.CompilerParams(use_tc_tiling_on_sc=False)


NBUF = 8
PF = 4


def _edge_sweep(tbl_hbm, gidx_v, sidx_v, rows_v, gsem, ssem, acc_s,
                cnt_fire=None):
    def g_start(j, b):
        pltpu.async_copy(tbl_hbm.at[gidx_v.at[j]], rows_v.at[b], gsem.at[b])

    def g_wait(b):
        pltpu.make_async_copy(tbl_hbm.at[gidx_v.at[0]], rows_v.at[b],
                              gsem.at[b]).wait()

    def s_start(j, b):
        pltpu.async_copy(rows_v.at[b], acc_s.at[sidx_v.at[j]], ssem.at[b],
                         add=True)

    def s_wait(b):
        pltpu.make_async_copy(rows_v.at[b], acc_s.at[sidx_v.at[0]],
                              ssem.at[b]).wait()

    def step(j, b, prefetch, wait_prev_scatter):
        g_wait(b)
        s_start(j, b)
        if cnt_fire is not None:
            cnt_fire(j)
        if prefetch:
            bp = (b + PF) % NBUF
            if wait_prev_scatter:
                s_wait(bp)
            g_start(j + PF, bp)

    for b in range(PF):
        g_start(b, b)
    for j in range(NBUF):
        step(j, j, prefetch=True, wait_prev_scatter=(j + PF >= NBUF))

    @pl.loop(NBUF, NB - NBUF, step=NBUF)
    def _(j0):
        for b in range(NBUF):
            step(j0 + b, b, prefetch=True, wait_prev_scatter=True)

    for j in range(NB - NBUF, NB):
        step(j, j % NBUF, prefetch=(j + PF < NB), wait_prev_scatter=True)
    for b in range(NBUF):
        s_wait(b)


def _sc_pass1_body(gidx_hbm, sidx_hbm, tbl_hbm, acc_out, cnt_out,
                   gidx_v, sidx_v, rows_v, ones_v, zbuf, zcnt, acc_s, cnt_s,
                   gsem, ssem, csem):
    cid = lax.axis_index("c")
    sid = lax.axis_index("s")
    tid = cid * 16 + sid

    @pl.loop(0, ZB)
    def _(i):
        zbuf[i] = jnp.zeros((16,), jnp.float32)

    @pl.loop(0, ZB, step=16)
    def _(i):
        zcnt[pl.ds(i, 16)] = jnp.zeros((16,), jnp.float32)

    for i in range(BT // 16):
        ones_v[pl.ds(i * 16, 16)] = jnp.ones((16,), jnp.float32)

    row0 = sid * ROWS_PT
    for k in range(ROWS_PT // ZB):
        pltpu.sync_copy(zbuf, acc_s.at[pl.ds(row0 + k * ZB, ZB)])
        pltpu.sync_copy(zcnt, cnt_s.at[pl.ds(row0 + k * ZB, ZB)])
    pltpu.sync_copy(gidx_hbm.at[pl.ds(tid * NB, NB)], gidx_v)
    pltpu.sync_copy(sidx_hbm.at[pl.ds(tid * NB, NB)], sidx_v)
    plsc.subcore_barrier()

    def cnt_fire(j):
        pltpu.async_copy(ones_v, cnt_s.at[sidx_v.at[j]], csem, add=True)

    _edge_sweep(tbl_hbm, gidx_v, sidx_v, rows_v, gsem, ssem, acc_s, cnt_fire)

    @pl.loop(0, NB)
    def _(_j):
        pltpu.make_async_copy(ones_v, cnt_s.at[sidx_v.at[0]], csem).wait()

    plsc.subcore_barrier()
    pltpu.sync_copy(acc_s.at[pl.ds(row0, ROWS_PT)],
                    acc_out.at[cid, pl.ds(row0, ROWS_PT)])
    pltpu.sync_copy(cnt_s.at[pl.ds(row0, ROWS_PT)],
                    cnt_out.at[cid, pl.ds(row0, ROWS_PT)])


_sc_pass1 = functools.partial(
    pl.kernel,
    out_type=[jax.ShapeDtypeStruct((2, NR, 16), jnp.float32),
              jax.ShapeDtypeStruct((2, NR), jnp.float32)],
    mesh=_mesh,
    scratch_types=[
        pltpu.VMEM((NB, BT), jnp.int32),
        pltpu.VMEM((NB, BT), jnp.int32),
        pltpu.VMEM((NBUF, BT, 16), jnp.float32),
        pltpu.VMEM((BT,), jnp.float32),
        pltpu.VMEM((ZB, 16), jnp.float32),
        pltpu.VMEM((ZB,), jnp.float32),
        pltpu.VMEM_SHARED((NR, 16), jnp.float32),
        pltpu.VMEM_SHARED((NR,), jnp.float32),
        pltpu.SemaphoreType.DMA((NBUF,)),
        pltpu.SemaphoreType.DMA((NBUF,)),
        pltpu.SemaphoreType.DMA,
    ],
    compiler_params=_sc_params,
)(_sc_pass1_body)


def _sc_pass2_body(gidx_hbm, sidx_hbm, tbl_hbm, acc_out,
                   gidx_v, sidx_v, rows_v, zbuf, acc_s, gsem, ssem):
    cid = lax.axis_index("c")
    sid = lax.axis_index("s")
    tid = cid * 16 + sid

    @pl.loop(0, ZB)
    def _(i):
        zbuf[i] = jnp.zeros((16,), jnp.float32)

    row0 = sid * ROWS_PT
    for k in range(ROWS_PT // ZB):
        pltpu.sync_copy(zbuf, acc_s.at[pl.ds(row0 + k * ZB, ZB)])
    pltpu.sync_copy(gidx_hbm.at[pl.ds(tid * NB, NB)], gidx_v)
    pltpu.sync_copy(sidx_hbm.at[pl.ds(tid * NB, NB)], sidx_v)
    plsc.subcore_barrier()

    _edge_sweep(tbl_hbm, gidx_v, sidx_v, rows_v, gsem, ssem, acc_s)

    plsc.subcore_barrier()
    pltpu.sync_copy(acc_s.at[pl.ds(row0, ROWS_PT)],
                    acc_out.at[cid, pl.ds(row0, ROWS_PT)])


_sc_pass2 = functools.partial(
    pl.kernel,
    out_type=jax.ShapeDtypeStruct((2, NR, 16), jnp.float32),
    mesh=_mesh,
    scratch_types=[
        pltpu.VMEM((NB, BT), jnp.int32),
        pltpu.VMEM((NB, BT), jnp.int32),
        pltpu.VMEM((NBUF, BT, 16), jnp.float32),
        pltpu.VMEM((ZB, 16), jnp.float32),
        pltpu.VMEM_SHARED((NR, 16), jnp.float32),
        pltpu.SemaphoreType.DMA((NBUF,)),
        pltpu.SemaphoreType.DMA((NBUF,)),
    ],
    compiler_params=_sc_params,
)(_sc_pass2_body)


EB_ROWS = E // BT
PAD_ROWS = EP // BT - EB_ROWS


def _idx_body(src_ref, dst_ref, et_ref, g_ref, s_ref, sp_ref):
    et = et_ref[...]
    g_ref[0:EB_ROWS] = src_ref[...] * R + et
    s_ref[0:EB_ROWS] = dst_ref[...] * R + et
    sp_ref[0:EB_ROWS] = src_ref[...]
    pidx = (lax.broadcasted_iota(jnp.int32, (PAD_ROWS, BT), 0) * BT
            + lax.broadcasted_iota(jnp.int32, (PAD_ROWS, BT), 1))
    spread = pidx % (NR - N * R)
    g_ref[EB_ROWS:] = spread
    s_ref[EB_ROWS:] = N * R + spread
    sp_ref[EB_ROWS:] = spread


_idx_call = pl.pallas_call(
    _idx_body,
    out_shape=[jax.ShapeDtypeStruct((EP // BT, BT), jnp.int32),
               jax.ShapeDtypeStruct((EP // BT, BT), jnp.int32),
               jax.ShapeDtypeStruct((EP // BT, BT), jnp.int32)],
)

BN = 1000


def _prep_body(x_ref, w_ref, r_ref, b_ref, xp_ref, o1_ref):
    xb = x_ref[...]
    xp_ref[...] = jnp.dot(xb, w_ref[...], preferred_element_type=jnp.float32)
    o1_ref[...] = jnp.dot(xb, r_ref[...],
                          preferred_element_type=jnp.float32) + b_ref[...]


_prep_call = pl.pallas_call(
    _prep_body,
    grid=(N // BN,),
    in_specs=[pl.BlockSpec((BN, 128), lambda i: (i, 0)),
              pl.BlockSpec((128, 128), lambda i: (0, 0)),
              pl.BlockSpec((128, 16), lambda i: (0, 0)),
              pl.BlockSpec((1, 16), lambda i: (0, 0))],
    out_specs=[pl.BlockSpec((BN, 128), lambda i: (i, 0)),
               pl.BlockSpec((BN, 16), lambda i: (i, 0))],
    out_shape=[jax.ShapeDtypeStruct((N, 128), jnp.float32),
               jax.ShapeDtypeStruct((N, 16), jnp.float32)],
)


def _mid_body(acc_ref, cnt_ref, o1_ref, b8_ref, s16_ref, h_ref):
    acc = acc_ref[0] + acc_ref[1]
    cnt = cnt_ref[0] + cnt_ref[1]
    cntw = jnp.dot(cnt, b8_ref[...], preferred_element_type=jnp.float32)
    inv = 1.0 / jnp.maximum(cntw, 1.0)
    hh = o1_ref[...] + jnp.dot(acc * inv, s16_ref[...],
                               preferred_element_type=jnp.float32)
    h_ref[...] = jnp.maximum(hh, 0.0)


_mid_call = pl.pallas_call(
    _mid_body,
    grid=(N // BN,),
    in_specs=[pl.BlockSpec((2, BN, 128), lambda i: (0, i, 0)),
              pl.BlockSpec((2, BN, 8), lambda i: (0, i, 0)),
              pl.BlockSpec((BN, 16), lambda i: (i, 0)),
              pl.BlockSpec((8, 128), lambda i: (0, 0)),
              pl.BlockSpec((128, 16), lambda i: (0, 0))],
    out_specs=pl.BlockSpec((BN, 16), lambda i: (i, 0)),
    out_shape=jax.ShapeDtypeStruct((N, 16), jnp.float32),
)


def _fin_body(acc_ref, cnt_ref, h_ref, r2_ref, b2_ref, b8_ref, w_ref,
              out_ref):
    acc = acc_ref[0] + acc_ref[1]
    cnt = cnt_ref[0] + cnt_ref[1]
    cntw = jnp.dot(cnt, b8_ref[...], preferred_element_type=jnp.float32)
    inv = 1.0 / jnp.maximum(cntw, 1.0)
    out_ref[...] = (jnp.dot(h_ref[...], r2_ref[...],
                            preferred_element_type=jnp.float32) + b2_ref[...]
                    + jnp.dot(acc * inv, w_ref[...],
                              preferred_element_type=jnp.float32))


_fin_call = pl.pallas_call(
    _fin_body,
    grid=(N // BN,),
    in_specs=[pl.BlockSpec((2, BN, 128), lambda i: (0, i, 0)),
              pl.BlockSpec((2, BN, 8), lambda i: (0, i, 0)),
              pl.BlockSpec((BN, 16), lambda i: (i, 0)),
              pl.BlockSpec((16, 128), lambda i: (0, 0)),
              pl.BlockSpec((1, 128), lambda i: (0, 0)),
              pl.BlockSpec((8, 128), lambda i: (0, 0)),
              pl.BlockSpec((128, 128), lambda i: (0, 0))],
    out_specs=pl.BlockSpec((BN, 128), lambda i: (i, 0)),
    out_shape=jax.ShapeDtypeStruct((N, 128), jnp.float32),
)


def kernel(x, edge_index, edge_type, W1, root1, b1, W2, root2, b2):
    src2 = edge_index[0].reshape(EB_ROWS, BT)
    dst2 = edge_index[1].reshape(EB_ROWS, BT)
    et2 = edge_type.reshape(EB_ROWS, BT)
    gidx2, sidx2, srcp2 = _idx_call(src2, dst2, et2)

    W1cat = W1.transpose(1, 0, 2).reshape(128, 128)
    xproj, o1root = _prep_call(x, W1cat, root1, b1.reshape(1, 16))

    acc1, cnt = _sc_pass1(gidx2, sidx2, xproj.reshape(N * R, 16))

    B8 = jnp.repeat(jnp.eye(R, dtype=jnp.float32), 16, axis=1)
    S16 = jnp.tile(jnp.eye(16, dtype=jnp.float32), (R, 1))
    cntr = cnt.reshape(2, NP_, R)
    h = _mid_call(acc1.reshape(2, NP_, 128), cntr, o1root, B8, S16)

    acc2 = _sc_pass2(srcp2, sidx2, h)
    out = _fin_call(acc2.reshape(2, NP_, 128), cntr, h, root2,
                    b2.reshape(1, 128), B8, W2.reshape(128, 128))
    return out


# speedup vs baseline: 1.8860x; 1.0299x over previous
import functools

import jax
import jax.numpy as jnp
from jax import lax
from jax.experimental import pallas as pl
from jax.experimental.pallas import tpu as pltpu
from jax.experimental.pallas import tpu_sc as plsc

N = 10000
E = 320000
R = 8
NP_ = 10240
NR = NP_ * R
NTILES = 32
EP = 327680
BT = 128
NB = EP // (NTILES * BT)
ROWS_PT = NR // 16
ZB = 128

_mesh = plsc.VectorSubcoreMesh(core_axis_name="c", subcore_axis_name="s")
_sc_params = pltpu.CompilerParams(use_tc_tiling_on_sc=False)


NBUF = 10
PF = 5


def _edge_sweep(tbl_hbm, gidx_v, sidx_v, rows_v, gsem, ssem, acc_s,
                cnt_fire=None):
    def g_start(j, b):
        pltpu.async_copy(tbl_hbm.at[gidx_v.at[j]], rows_v.at[b], gsem.at[b])

    def g_wait(b):
        pltpu.make_async_copy(tbl_hbm.at[gidx_v.at[0]], rows_v.at[b],
                              gsem.at[b]).wait()

    def s_start(j, b):
        pltpu.async_copy(rows_v.at[b], acc_s.at[sidx_v.at[j]], ssem.at[b],
                         add=True)

    def s_wait(b):
        pltpu.make_async_copy(rows_v.at[b], acc_s.at[sidx_v.at[0]],
                              ssem.at[b]).wait()

    def step(j, b, prefetch, wait_prev_scatter):
        g_wait(b)
        s_start(j, b)
        if cnt_fire is not None:
            cnt_fire(j)
        if prefetch:
            bp = (b + PF) % NBUF
            if wait_prev_scatter:
                s_wait(bp)
            g_start(j + PF, bp)

    for b in range(PF):
        g_start(b, b)
    for j in range(NBUF):
        step(j, j, prefetch=True, wait_prev_scatter=(j + PF >= NBUF))

    @pl.loop(NBUF, NB - NBUF, step=NBUF)
    def _(j0):
        for b in range(NBUF):
            step(j0 + b, b, prefetch=True, wait_prev_scatter=True)

    for j in range(NB - NBUF, NB):
        step(j, j % NBUF, prefetch=(j + PF < NB), wait_prev_scatter=True)
    for b in range(NBUF):
        s_wait(b)


def _sc_pass1_body(gidx_hbm, sidx_hbm, tbl_hbm, acc_out, cnt_out,
                   gidx_v, sidx_v, rows_v, ones_v, zbuf, zcnt, acc_s, cnt_s,
                   gsem, ssem, csem):
    cid = lax.axis_index("c")
    sid = lax.axis_index("s")
    tid = cid * 16 + sid

    @pl.loop(0, ZB)
    def _(i):
        zbuf[i] = jnp.zeros((16,), jnp.float32)

    @pl.loop(0, ZB, step=16)
    def _(i):
        zcnt[pl.ds(i, 16)] = jnp.zeros((16,), jnp.float32)

    for i in range(BT // 16):
        ones_v[pl.ds(i * 16, 16)] = jnp.ones((16,), jnp.float32)

    row0 = sid * ROWS_PT
    for k in range(ROWS_PT // ZB):
        pltpu.sync_copy(zbuf, acc_s.at[pl.ds(row0 + k * ZB, ZB)])
        pltpu.sync_copy(zcnt, cnt_s.at[pl.ds(row0 + k * ZB, ZB)])
    pltpu.sync_copy(gidx_hbm.at[pl.ds(tid * NB, NB)], gidx_v)
    pltpu.sync_copy(sidx_hbm.at[pl.ds(tid * NB, NB)], sidx_v)
    plsc.subcore_barrier()

    def cnt_fire(j):
        pltpu.async_copy(ones_v, cnt_s.at[sidx_v.at[j]], csem, add=True)

    _edge_sweep(tbl_hbm, gidx_v, sidx_v, rows_v, gsem, ssem, acc_s, cnt_fire)

    @pl.loop(0, NB)
    def _(_j):
        pltpu.make_async_copy(ones_v, cnt_s.at[sidx_v.at[0]], csem).wait()

    plsc.subcore_barrier()
    pltpu.sync_copy(acc_s.at[pl.ds(row0, ROWS_PT)],
                    acc_out.at[cid, pl.ds(row0, ROWS_PT)])
    pltpu.sync_copy(cnt_s.at[pl.ds(row0, ROWS_PT)],
                    cnt_out.at[cid, pl.ds(row0, ROWS_PT)])


_sc_pass1 = functools.partial(
    pl.kernel,
    out_type=[jax.ShapeDtypeStruct((2, NR, 16), jnp.float32),
              jax.ShapeDtypeStruct((2, NR), jnp.float32)],
    mesh=_mesh,
    scratch_types=[
        pltpu.VMEM((NB, BT), jnp.int32),
        pltpu.VMEM((NB, BT), jnp.int32),
        pltpu.VMEM((NBUF, BT, 16), jnp.float32),
        pltpu.VMEM((BT,), jnp.float32),
        pltpu.VMEM((ZB, 16), jnp.float32),
        pltpu.VMEM((ZB,), jnp.float32),
        pltpu.VMEM_SHARED((NR, 16), jnp.float32),
        pltpu.VMEM_SHARED((NR,), jnp.float32),
        pltpu.SemaphoreType.DMA((NBUF,)),
        pltpu.SemaphoreType.DMA((NBUF,)),
        pltpu.SemaphoreType.DMA,
    ],
    compiler_params=_sc_params,
)(_sc_pass1_body)


def _sc_pass2_body(gidx_hbm, sidx_hbm, tbl_hbm, acc_out,
                   gidx_v, sidx_v, rows_v, zbuf, acc_s, gsem, ssem):
    cid = lax.axis_index("c")
    sid = lax.axis_index("s")
    tid = cid * 16 + sid

    @pl.loop(0, ZB)
    def _(i):
        zbuf[i] = jnp.zeros((16,), jnp.float32)

    row0 = sid * ROWS_PT
    for k in range(ROWS_PT // ZB):
        pltpu.sync_copy(zbuf, acc_s.at[pl.ds(row0 + k * ZB, ZB)])
    pltpu.sync_copy(gidx_hbm.at[pl.ds(tid * NB, NB)], gidx_v)
    pltpu.sync_copy(sidx_hbm.at[pl.ds(tid * NB, NB)], sidx_v)
    plsc.subcore_barrier()

    _edge_sweep(tbl_hbm, gidx_v, sidx_v, rows_v, gsem, ssem, acc_s)

    plsc.subcore_barrier()
    pltpu.sync_copy(acc_s.at[pl.ds(row0, ROWS_PT)],
                    acc_out.at[cid, pl.ds(row0, ROWS_PT)])


_sc_pass2 = functools.partial(
    pl.kernel,
    out_type=jax.ShapeDtypeStruct((2, NR, 16), jnp.float32),
    mesh=_mesh,
    scratch_types=[
        pltpu.VMEM((NB, BT), jnp.int32),
        pltpu.VMEM((NB, BT), jnp.int32),
        pltpu.VMEM((NBUF, BT, 16), jnp.float32),
        pltpu.VMEM((ZB, 16), jnp.float32),
        pltpu.VMEM_SHARED((NR, 16), jnp.float32),
        pltpu.SemaphoreType.DMA((NBUF,)),
        pltpu.SemaphoreType.DMA((NBUF,)),
    ],
    compiler_params=_sc_params,
)(_sc_pass2_body)


EB_ROWS = E // BT
PAD_ROWS = EP // BT - EB_ROWS


def _idx_body(src_ref, dst_ref, et_ref, g_ref, s_ref, sp_ref):
    et = et_ref[...]
    g_ref[0:EB_ROWS] = src_ref[...] * R + et
    s_ref[0:EB_ROWS] = dst_ref[...] * R + et
    sp_ref[0:EB_ROWS] = src_ref[...]
    pidx = (lax.broadcasted_iota(jnp.int32, (PAD_ROWS, BT), 0) * BT
            + lax.broadcasted_iota(jnp.int32, (PAD_ROWS, BT), 1))
    spread = pidx % (NR - N * R)
    g_ref[EB_ROWS:] = spread
    s_ref[EB_ROWS:] = N * R + spread
    sp_ref[EB_ROWS:] = spread


_idx_call = pl.pallas_call(
    _idx_body,
    out_shape=[jax.ShapeDtypeStruct((EP // BT, BT), jnp.int32),
               jax.ShapeDtypeStruct((EP // BT, BT), jnp.int32),
               jax.ShapeDtypeStruct((EP // BT, BT), jnp.int32)],
)

BN = 1000


def _prep_body(x_ref, w_ref, r_ref, b_ref, xp_ref, o1_ref):
    xb = x_ref[...]
    xp_ref[...] = jnp.dot(xb, w_ref[...], preferred_element_type=jnp.float32)
    o1_ref[...] = jnp.dot(xb, r_ref[...],
                          preferred_element_type=jnp.float32) + b_ref[...]


_prep_call = pl.pallas_call(
    _prep_body,
    grid=(N // BN,),
    in_specs=[pl.BlockSpec((BN, 128), lambda i: (i, 0)),
              pl.BlockSpec((128, 128), lambda i: (0, 0)),
              pl.BlockSpec((128, 16), lambda i: (0, 0)),
              pl.BlockSpec((1, 16), lambda i: (0, 0))],
    out_specs=[pl.BlockSpec((BN, 128), lambda i: (i, 0)),
               pl.BlockSpec((BN, 16), lambda i: (i, 0))],
    out_shape=[jax.ShapeDtypeStruct((N, 128), jnp.float32),
               jax.ShapeDtypeStruct((N, 16), jnp.float32)],
)


def _mid_body(acc_ref, cnt_ref, o1_ref, b8_ref, s16_ref, h_ref):
    acc = acc_ref[0] + acc_ref[1]
    cnt = cnt_ref[0] + cnt_ref[1]
    cntw = jnp.dot(cnt, b8_ref[...], preferred_element_type=jnp.float32)
    inv = 1.0 / jnp.maximum(cntw, 1.0)
    hh = o1_ref[...] + jnp.dot(acc * inv, s16_ref[...],
                               preferred_element_type=jnp.float32)
    h_ref[...] = jnp.maximum(hh, 0.0)


_mid_call = pl.pallas_call(
    _mid_body,
    grid=(N // BN,),
    in_specs=[pl.BlockSpec((2, BN, 128), lambda i: (0, i, 0)),
              pl.BlockSpec((2, BN, 8), lambda i: (0, i, 0)),
              pl.BlockSpec((BN, 16), lambda i: (i, 0)),
              pl.BlockSpec((8, 128), lambda i: (0, 0)),
              pl.BlockSpec((128, 16), lambda i: (0, 0))],
    out_specs=pl.BlockSpec((BN, 16), lambda i: (i, 0)),
    out_shape=jax.ShapeDtypeStruct((N, 16), jnp.float32),
)


def _fin_body(acc_ref, cnt_ref, h_ref, r2_ref, b2_ref, b8_ref, w_ref,
              out_ref):
    acc = acc_ref[0] + acc_ref[1]
    cnt = cnt_ref[0] + cnt_ref[1]
    cntw = jnp.dot(cnt, b8_ref[...], preferred_element_type=jnp.float32)
    inv = 1.0 / jnp.maximum(cntw, 1.0)
    out_ref[...] = (jnp.dot(h_ref[...], r2_ref[...],
                            preferred_element_type=jnp.float32) + b2_ref[...]
                    + jnp.dot(acc * inv, w_ref[...],
                              preferred_element_type=jnp.float32))


_fin_call = pl.pallas_call(
    _fin_body,
    grid=(N // BN,),
    in_specs=[pl.BlockSpec((2, BN, 128), lambda i: (0, i, 0)),
              pl.BlockSpec((2, BN, 8), lambda i: (0, i, 0)),
              pl.BlockSpec((BN, 16), lambda i: (i, 0)),
              pl.BlockSpec((16, 128), lambda i: (0, 0)),
              pl.BlockSpec((1, 128), lambda i: (0, 0)),
              pl.BlockSpec((8, 128), lambda i: (0, 0)),
              pl.BlockSpec((128, 128), lambda i: (0, 0))],
    out_specs=pl.BlockSpec((BN, 128), lambda i: (i, 0)),
    out_shape=jax.ShapeDtypeStruct((N, 128), jnp.float32),
)


def kernel(x, edge_index, edge_type, W1, root1, b1, W2, root2, b2):
    src2 = edge_index[0].reshape(EB_ROWS, BT)
    dst2 = edge_index[1].reshape(EB_ROWS, BT)
    et2 = edge_type.reshape(EB_ROWS, BT)
    gidx2, sidx2, srcp2 = _idx_call(src2, dst2, et2)

    W1cat = W1.transpose(1, 0, 2).reshape(128, 128)
    xproj, o1root = _prep_call(x, W1cat, root1, b1.reshape(1, 16))

    acc1, cnt = _sc_pass1(gidx2, sidx2, xproj.reshape(N * R, 16))

    B8 = jnp.repeat(jnp.eye(R, dtype=jnp.float32), 16, axis=1)
    S16 = jnp.tile(jnp.eye(16, dtype=jnp.float32), (R, 1))
    cntr = cnt.reshape(2, NP_, R)
    h = _mid_call(acc1.reshape(2, NP_, 128), cntr, o1root, B8, S16)

    acc2 = _sc_pass2(srcp2, sidx2, h)
    out = _fin_call(acc2.reshape(2, NP_, 128), cntr, h, root2,
                    b2.reshape(1, 128), B8, W2.reshape(128, 128))
    return out


# speedup vs baseline: 1.9865x; 1.0533x over previous
import functools

import jax
import jax.numpy as jnp
from jax import lax
from jax.experimental import pallas as pl
from jax.experimental.pallas import tpu as pltpu
from jax.experimental.pallas import tpu_sc as plsc

N = 10000
E = 320000
R = 8
NP_ = 10240
NR = NP_ * R
NTILES = 32
EP = 327680
BT = 128
NB = EP // (NTILES * BT)
ROWS_PT = NR // 16
ZB = 128

_mesh = plsc.VectorSubcoreMesh(core_axis_name="c", subcore_axis_name="s")
_sc_params = pltpu.CompilerParams(use_tc_tiling_on_sc=False)


NBUF = 10
PF = 5


def _edge_sweep(tbl_hbm, gidx_v, sidx_v, rows_v, gsem, ssem, acc_s,
                cnt_fire=None):
    def g_start(j, b):
        pltpu.async_copy(tbl_hbm.at[gidx_v.at[j]], rows_v.at[b], gsem.at[b])

    def g_wait(b):
        pltpu.make_async_copy(tbl_hbm.at[gidx_v.at[0]], rows_v.at[b],
                              gsem.at[b]).wait()

    def s_start(j, b):
        pltpu.async_copy(rows_v.at[b], acc_s.at[sidx_v.at[j]], ssem.at[b],
                         add=True)

    def s_wait(b):
        pltpu.make_async_copy(rows_v.at[b], acc_s.at[sidx_v.at[0]],
                              ssem.at[b]).wait()

    def step(j, b, prefetch, wait_prev_scatter):
        g_wait(b)
        s_start(j, b)
        if cnt_fire is not None:
            cnt_fire(j)
        if prefetch:
            bp = (b + PF) % NBUF
            if wait_prev_scatter:
                s_wait(bp)
            g_start(j + PF, bp)

    for b in range(PF):
        g_start(b, b)
    for j in range(NBUF):
        step(j, j, prefetch=True, wait_prev_scatter=(j + PF >= NBUF))

    @pl.loop(NBUF, NB - NBUF, step=NBUF)
    def _(j0):
        for b in range(NBUF):
            step(j0 + b, b, prefetch=True, wait_prev_scatter=True)

    for j in range(NB - NBUF, NB):
        step(j, j % NBUF, prefetch=(j + PF < NB), wait_prev_scatter=True)
    for b in range(NBUF):
        s_wait(b)


def _sc_pass1_body(gidx_hbm, sidx_hbm, tbl_hbm, acc_out, cnt_out,
                   gidx_v, sidx_v, rows_v, ones_v, zbuf, zcnt, acc_s, cnt_s,
                   gsem, ssem, csem):
    cid = lax.axis_index("c")
    sid = lax.axis_index("s")
    tid = cid * 16 + sid

    @pl.loop(0, ZB)
    def _(i):
        zbuf[i] = jnp.zeros((16,), jnp.float32)

    @pl.loop(0, ZB, step=16)
    def _(i):
        zcnt[pl.ds(i, 16)] = jnp.zeros((16,), jnp.float32)

    for i in range(BT // 16):
        ones_v[pl.ds(i * 16, 16)] = jnp.ones((16,), jnp.float32)

    row0 = sid * ROWS_PT
    for k in range(ROWS_PT // ZB):
        pltpu.sync_copy(zbuf, acc_s.at[pl.ds(row0 + k * ZB, ZB)])
        pltpu.sync_copy(zcnt, cnt_s.at[pl.ds(row0 + k * ZB, ZB)])
    pltpu.sync_copy(gidx_hbm.at[pl.ds(tid * NB, NB)], gidx_v)
    pltpu.sync_copy(sidx_hbm.at[pl.ds(tid * NB, NB)], sidx_v)
    plsc.subcore_barrier()

    def cnt_fire(j):
        pltpu.async_copy(ones_v, cnt_s.at[sidx_v.at[j]], csem, add=True)

    _edge_sweep(tbl_hbm, gidx_v, sidx_v, rows_v, gsem, ssem, acc_s, cnt_fire)

    @pl.loop(0, NB)
    def _(_j):
        pltpu.make_async_copy(ones_v, cnt_s.at[sidx_v.at[0]], csem).wait()

    plsc.subcore_barrier()
    pltpu.sync_copy(acc_s.at[pl.ds(row0, ROWS_PT)],
                    acc_out.at[cid, pl.ds(row0, ROWS_PT)])
    pltpu.sync_copy(cnt_s.at[pl.ds(row0, ROWS_PT)],
                    cnt_out.at[cid, pl.ds(row0, ROWS_PT)])


_sc_pass1 = functools.partial(
    pl.kernel,
    out_type=[jax.ShapeDtypeStruct((2, NR, 16), jnp.float32),
              jax.ShapeDtypeStruct((2, NR), jnp.float32)],
    mesh=_mesh,
    scratch_types=[
        pltpu.VMEM((NB, BT), jnp.int32),
        pltpu.VMEM((NB, BT), jnp.int32),
        pltpu.VMEM((NBUF, BT, 16), jnp.float32),
        pltpu.VMEM((BT,), jnp.float32),
        pltpu.VMEM((ZB, 16), jnp.float32),
        pltpu.VMEM((ZB,), jnp.float32),
        pltpu.VMEM_SHARED((NR, 16), jnp.float32),
        pltpu.VMEM_SHARED((NR,), jnp.float32),
        pltpu.SemaphoreType.DMA((NBUF,)),
        pltpu.SemaphoreType.DMA((NBUF,)),
        pltpu.SemaphoreType.DMA,
    ],
    compiler_params=_sc_params,
)(_sc_pass1_body)


def _sc_pass2_body(gidx_hbm, sidx_hbm, tbl_hbm, acc_out,
                   gidx_v, sidx_v, rows_v, zbuf, acc_s, gsem, ssem):
    cid = lax.axis_index("c")
    sid = lax.axis_index("s")
    tid = cid * 16 + sid

    @pl.loop(0, ZB)
    def _(i):
        zbuf[i] = jnp.zeros((16,), jnp.float32)

    row0 = sid * ROWS_PT
    for k in range(ROWS_PT // ZB):
        pltpu.sync_copy(zbuf, acc_s.at[pl.ds(row0 + k * ZB, ZB)])
    pltpu.sync_copy(gidx_hbm.at[pl.ds(tid * NB, NB)], gidx_v)
    pltpu.sync_copy(sidx_hbm.at[pl.ds(tid * NB, NB)], sidx_v)
    plsc.subcore_barrier()

    _edge_sweep(tbl_hbm, gidx_v, sidx_v, rows_v, gsem, ssem, acc_s)

    plsc.subcore_barrier()
    pltpu.sync_copy(acc_s.at[pl.ds(row0, ROWS_PT)],
                    acc_out.at[cid, pl.ds(row0, ROWS_PT)])


_sc_pass2 = functools.partial(
    pl.kernel,
    out_type=jax.ShapeDtypeStruct((2, NR, 16), jnp.float32),
    mesh=_mesh,
    scratch_types=[
        pltpu.VMEM((NB, BT), jnp.int32),
        pltpu.VMEM((NB, BT), jnp.int32),
        pltpu.VMEM((NBUF, BT, 16), jnp.float32),
        pltpu.VMEM((ZB, 16), jnp.float32),
        pltpu.VMEM_SHARED((NR, 16), jnp.float32),
        pltpu.SemaphoreType.DMA((NBUF,)),
        pltpu.SemaphoreType.DMA((NBUF,)),
    ],
    compiler_params=_sc_params,
)(_sc_pass2_body)


EB_ROWS = E // BT
PAD_ROWS = EP // BT - EB_ROWS


def _idx_body(src_ref, dst_ref, et_ref, g_ref, s_ref, sp_ref):
    et = et_ref[...]
    g_ref[0:EB_ROWS] = src_ref[...] * R + et
    s_ref[0:EB_ROWS] = dst_ref[...] * R + et
    sp_ref[0:EB_ROWS] = src_ref[...]
    pidx = (lax.broadcasted_iota(jnp.int32, (PAD_ROWS, BT), 0) * BT
            + lax.broadcasted_iota(jnp.int32, (PAD_ROWS, BT), 1))
    spread = pidx % (NR - N * R)
    g_ref[EB_ROWS:] = spread
    s_ref[EB_ROWS:] = N * R + spread
    sp_ref[EB_ROWS:] = spread


_idx_call = pl.pallas_call(
    _idx_body,
    out_shape=[jax.ShapeDtypeStruct((EP // BT, BT), jnp.int32),
               jax.ShapeDtypeStruct((EP // BT, BT), jnp.int32),
               jax.ShapeDtypeStruct((EP // BT, BT), jnp.int32)],
)

BN = 2000


def _prep_body(x_ref, w_ref, r_ref, b_ref, xp_ref, o1_ref):
    xb = x_ref[...]
    xp_ref[...] = jnp.dot(xb, w_ref[...], preferred_element_type=jnp.float32)
    o1_ref[...] = jnp.dot(xb, r_ref[...],
                          preferred_element_type=jnp.float32) + b_ref[...]


_prep_call = pl.pallas_call(
    _prep_body,
    grid=(N // BN,),
    in_specs=[pl.BlockSpec((BN, 128), lambda i: (i, 0)),
              pl.BlockSpec((128, 128), lambda i: (0, 0)),
              pl.BlockSpec((128, 16), lambda i: (0, 0)),
              pl.BlockSpec((1, 16), lambda i: (0, 0))],
    out_specs=[pl.BlockSpec((BN, 128), lambda i: (i, 0)),
               pl.BlockSpec((BN, 16), lambda i: (i, 0))],
    out_shape=[jax.ShapeDtypeStruct((N, 128), jnp.float32),
               jax.ShapeDtypeStruct((N, 16), jnp.float32)],
)


def _mid_body(acc_ref, cnt_ref, o1_ref, b8_ref, s16_ref, h_ref):
    acc = acc_ref[0] + acc_ref[1]
    cnt = cnt_ref[0] + cnt_ref[1]
    cntw = jnp.dot(cnt, b8_ref[...], preferred_element_type=jnp.float32)
    inv = 1.0 / jnp.maximum(cntw, 1.0)
    hh = o1_ref[...] + jnp.dot(acc * inv, s16_ref[...],
                               preferred_element_type=jnp.float32)
    h_ref[...] = jnp.maximum(hh, 0.0)


_mid_call = pl.pallas_call(
    _mid_body,
    grid=(N // BN,),
    in_specs=[pl.BlockSpec((2, BN, 128), lambda i: (0, i, 0)),
              pl.BlockSpec((2, BN, 8), lambda i: (0, i, 0)),
              pl.BlockSpec((BN, 16), lambda i: (i, 0)),
              pl.BlockSpec((8, 128), lambda i: (0, 0)),
              pl.BlockSpec((128, 16), lambda i: (0, 0))],
    out_specs=pl.BlockSpec((BN, 16), lambda i: (i, 0)),
    out_shape=jax.ShapeDtypeStruct((N, 16), jnp.float32),
)


def _fin_body(acc_ref, cnt_ref, h_ref, r2_ref, b2_ref, b8_ref, w_ref,
              out_ref):
    acc = acc_ref[0] + acc_ref[1]
    cnt = cnt_ref[0] + cnt_ref[1]
    cntw = jnp.dot(cnt, b8_ref[...], preferred_element_type=jnp.float32)
    inv = 1.0 / jnp.maximum(cntw, 1.0)
    out_ref[...] = (jnp.dot(h_ref[...], r2_ref[...],
                            preferred_element_type=jnp.float32) + b2_ref[...]
                    + jnp.dot(acc * inv, w_ref[...],
                              preferred_element_type=jnp.float32))


_fin_call = pl.pallas_call(
    _fin_body,
    grid=(N // BN,),
    in_specs=[pl.BlockSpec((2, BN, 128), lambda i: (0, i, 0)),
              pl.BlockSpec((2, BN, 8), lambda i: (0, i, 0)),
              pl.BlockSpec((BN, 16), lambda i: (i, 0)),
              pl.BlockSpec((16, 128), lambda i: (0, 0)),
              pl.BlockSpec((1, 128), lambda i: (0, 0)),
              pl.BlockSpec((8, 128), lambda i: (0, 0)),
              pl.BlockSpec((128, 128), lambda i: (0, 0))],
    out_specs=pl.BlockSpec((BN, 128), lambda i: (i, 0)),
    out_shape=jax.ShapeDtypeStruct((N, 128), jnp.float32),
)


def kernel(x, edge_index, edge_type, W1, root1, b1, W2, root2, b2):
    src2 = edge_index[0].reshape(EB_ROWS, BT)
    dst2 = edge_index[1].reshape(EB_ROWS, BT)
    et2 = edge_type.reshape(EB_ROWS, BT)
    gidx2, sidx2, srcp2 = _idx_call(src2, dst2, et2)

    W1cat = W1.transpose(1, 0, 2).reshape(128, 128)
    xproj, o1root = _prep_call(x, W1cat, root1, b1.reshape(1, 16))

    acc1, cnt = _sc_pass1(gidx2, sidx2, xproj.reshape(N * R, 16))

    B8 = jnp.repeat(jnp.eye(R, dtype=jnp.float32), 16, axis=1)
    S16 = jnp.tile(jnp.eye(16, dtype=jnp.float32), (R, 1))
    cntr = cnt.reshape(2, NP_, R)
    h = _mid_call(acc1.reshape(2, NP_, 128), cntr, o1root, B8, S16)

    acc2 = _sc_pass2(srcp2, sidx2, h)
    out = _fin_call(acc2.reshape(2, NP_, 128), cntr, h, root2,
                    b2.reshape(1, 128), B8, W2.reshape(128, 128))
    return out


# speedup vs baseline: 2.1278x; 1.0711x over previous
import functools

import jax
import jax.numpy as jnp
from jax import lax
from jax.experimental import pallas as pl
from jax.experimental.pallas import tpu as pltpu
from jax.experimental.pallas import tpu_sc as plsc

N = 10000
E = 320000
R = 8
NP_ = 10240
NR = NP_ * R
NTILES = 32
EP = 327680
BT = 128
NB = EP // (NTILES * BT)
ROWS_PT = NR // 16
ZB = 128

_mesh = plsc.VectorSubcoreMesh(core_axis_name="c", subcore_axis_name="s")
_sc_params = pltpu.CompilerParams(use_tc_tiling_on_sc=False)


NBUF = 10
PF = 5


def _edge_sweep(tbl_hbm, gidx_v, sidx_v, rows_v, gsem, ssem, acc_s,
                cnt_fire=None):
    def g_start(j, b):
        pltpu.async_copy(tbl_hbm.at[gidx_v.at[j]], rows_v.at[b], gsem.at[b])

    def g_wait(b):
        pltpu.make_async_copy(tbl_hbm.at[gidx_v.at[0]], rows_v.at[b],
                              gsem.at[b]).wait()

    def s_start(j, b):
        pltpu.async_copy(rows_v.at[b], acc_s.at[sidx_v.at[j]], ssem.at[b],
                         add=True)

    def s_wait(b):
        pltpu.make_async_copy(rows_v.at[b], acc_s.at[sidx_v.at[0]],
                              ssem.at[b]).wait()

    def step(j, b, prefetch, wait_prev_scatter):
        g_wait(b)
        s_start(j, b)
        if cnt_fire is not None:
            cnt_fire(j)
        if prefetch:
            bp = (b + PF) % NBUF
            if wait_prev_scatter:
                s_wait(bp)
            g_start(j + PF, bp)

    for b in range(PF):
        g_start(b, b)
    for j in range(NBUF):
        step(j, j, prefetch=True, wait_prev_scatter=(j + PF >= NBUF))

    @pl.loop(NBUF, NB - NBUF, step=NBUF)
    def _(j0):
        for b in range(NBUF):
            step(j0 + b, b, prefetch=True, wait_prev_scatter=True)

    for j in range(NB - NBUF, NB):
        step(j, j % NBUF, prefetch=(j + PF < NB), wait_prev_scatter=True)
    for b in range(NBUF):
        s_wait(b)


def _sc_pass1_body(gidx_hbm, sidx_hbm, tbl_hbm, acc_out, cnt_out,
                   gidx_v, sidx_v, rows_v, ones_v, zbuf, zcnt, acc_s, cnt_s,
                   gsem, ssem, csem):
    cid = lax.axis_index("c")
    sid = lax.axis_index("s")
    tid = cid * 16 + sid

    @pl.loop(0, ZB)
    def _(i):
        zbuf[i] = jnp.zeros((16,), jnp.float32)

    @pl.loop(0, ZB, step=16)
    def _(i):
        zcnt[pl.ds(i, 16)] = jnp.zeros((16,), jnp.float32)

    for i in range(BT // 16):
        ones_v[pl.ds(i * 16, 16)] = jnp.ones((16,), jnp.float32)

    row0 = sid * ROWS_PT
    for k in range(ROWS_PT // ZB):
        pltpu.async_copy(zbuf, acc_s.at[pl.ds(row0 + k * ZB, ZB)], csem)
        pltpu.async_copy(zcnt, cnt_s.at[pl.ds(row0 + k * ZB, ZB)], csem)
    pltpu.async_copy(gidx_hbm.at[pl.ds(tid * NB, NB)], gidx_v, gsem.at[0])
    pltpu.async_copy(sidx_hbm.at[pl.ds(tid * NB, NB)], sidx_v, gsem.at[1])
    for k in range(ROWS_PT // ZB):
        pltpu.make_async_copy(zbuf, acc_s.at[pl.ds(row0, ZB)], csem).wait()
        pltpu.make_async_copy(zcnt, cnt_s.at[pl.ds(row0, ZB)], csem).wait()
    pltpu.make_async_copy(gidx_hbm.at[pl.ds(0, NB)], gidx_v, gsem.at[0]).wait()
    pltpu.make_async_copy(sidx_hbm.at[pl.ds(0, NB)], sidx_v, gsem.at[1]).wait()
    plsc.subcore_barrier()

    def cnt_fire(j):
        pltpu.async_copy(ones_v, cnt_s.at[sidx_v.at[j]], csem, add=True)

    _edge_sweep(tbl_hbm, gidx_v, sidx_v, rows_v, gsem, ssem, acc_s, cnt_fire)

    @pl.loop(0, NB)
    def _(_j):
        pltpu.make_async_copy(ones_v, cnt_s.at[sidx_v.at[0]], csem).wait()

    plsc.subcore_barrier()
    pltpu.sync_copy(acc_s.at[pl.ds(row0, ROWS_PT)],
                    acc_out.at[cid, pl.ds(row0, ROWS_PT)])
    pltpu.sync_copy(cnt_s.at[pl.ds(row0, ROWS_PT)],
                    cnt_out.at[cid, pl.ds(row0, ROWS_PT)])


_sc_pass1 = functools.partial(
    pl.kernel,
    out_type=[jax.ShapeDtypeStruct((2, NR, 16), jnp.float32),
              jax.ShapeDtypeStruct((2, NR), jnp.float32)],
    mesh=_mesh,
    scratch_types=[
        pltpu.VMEM((NB, BT), jnp.int32),
        pltpu.VMEM((NB, BT), jnp.int32),
        pltpu.VMEM((NBUF, BT, 16), jnp.float32),
        pltpu.VMEM((BT,), jnp.float32),
        pltpu.VMEM((ZB, 16), jnp.float32),
        pltpu.VMEM((ZB,), jnp.float32),
        pltpu.VMEM_SHARED((NR, 16), jnp.float32),
        pltpu.VMEM_SHARED((NR,), jnp.float32),
        pltpu.SemaphoreType.DMA((NBUF,)),
        pltpu.SemaphoreType.DMA((NBUF,)),
        pltpu.SemaphoreType.DMA,
    ],
    compiler_params=_sc_params,
)(_sc_pass1_body)


def _sc_pass2_body(gidx_hbm, sidx_hbm, tbl_hbm, acc_out,
                   gidx_v, sidx_v, rows_v, zbuf, acc_s, gsem, ssem):
    cid = lax.axis_index("c")
    sid = lax.axis_index("s")
    tid = cid * 16 + sid

    @pl.loop(0, ZB)
    def _(i):
        zbuf[i] = jnp.zeros((16,), jnp.float32)

    row0 = sid * ROWS_PT
    for k in range(ROWS_PT // ZB):
        pltpu.async_copy(zbuf, acc_s.at[pl.ds(row0 + k * ZB, ZB)], ssem.at[0])
    pltpu.async_copy(gidx_hbm.at[pl.ds(tid * NB, NB)], gidx_v, gsem.at[0])
    pltpu.async_copy(sidx_hbm.at[pl.ds(tid * NB, NB)], sidx_v, gsem.at[1])
    for k in range(ROWS_PT // ZB):
        pltpu.make_async_copy(zbuf, acc_s.at[pl.ds(row0, ZB)],
                              ssem.at[0]).wait()
    pltpu.make_async_copy(gidx_hbm.at[pl.ds(0, NB)], gidx_v, gsem.at[0]).wait()
    pltpu.make_async_copy(sidx_hbm.at[pl.ds(0, NB)], sidx_v, gsem.at[1]).wait()
    plsc.subcore_barrier()

    _edge_sweep(tbl_hbm, gidx_v, sidx_v, rows_v, gsem, ssem, acc_s)

    plsc.subcore_barrier()
    pltpu.sync_copy(acc_s.at[pl.ds(row0, ROWS_PT)],
                    acc_out.at[cid, pl.ds(row0, ROWS_PT)])


_sc_pass2 = functools.partial(
    pl.kernel,
    out_type=jax.ShapeDtypeStruct((2, NR, 16), jnp.float32),
    mesh=_mesh,
    scratch_types=[
        pltpu.VMEM((NB, BT), jnp.int32),
        pltpu.VMEM((NB, BT), jnp.int32),
        pltpu.VMEM((NBUF, BT, 16), jnp.float32),
        pltpu.VMEM((ZB, 16), jnp.float32),
        pltpu.VMEM_SHARED((NR, 16), jnp.float32),
        pltpu.SemaphoreType.DMA((NBUF,)),
        pltpu.SemaphoreType.DMA((NBUF,)),
    ],
    compiler_params=_sc_params,
)(_sc_pass2_body)


EB_ROWS = E // BT
PAD_ROWS = EP // BT - EB_ROWS


def _idx_body(src_ref, dst_ref, et_ref, g_ref, s_ref, sp_ref):
    et = et_ref[...]
    g_ref[0:EB_ROWS] = src_ref[...] * R + et
    s_ref[0:EB_ROWS] = dst_ref[...] * R + et
    sp_ref[0:EB_ROWS] = src_ref[...]
    pidx = (lax.broadcasted_iota(jnp.int32, (PAD_ROWS, BT), 0) * BT
            + lax.broadcasted_iota(jnp.int32, (PAD_ROWS, BT), 1))
    spread = pidx % (NR - N * R)
    g_ref[EB_ROWS:] = spread
    s_ref[EB_ROWS:] = N * R + spread
    sp_ref[EB_ROWS:] = spread


_idx_call = pl.pallas_call(
    _idx_body,
    out_shape=[jax.ShapeDtypeStruct((EP // BT, BT), jnp.int32),
               jax.ShapeDtypeStruct((EP // BT, BT), jnp.int32),
               jax.ShapeDtypeStruct((EP // BT, BT), jnp.int32)],
)

BN = 2000


def _prep_body(x_ref, w_ref, r_ref, b_ref, xp_ref, o1_ref):
    xb = x_ref[...]
    xp_ref[...] = jnp.dot(xb, w_ref[...], preferred_element_type=jnp.float32)
    o1_ref[...] = jnp.dot(xb, r_ref[...],
                          preferred_element_type=jnp.float32) + b_ref[...]


_prep_call = pl.pallas_call(
    _prep_body,
    grid=(N // BN,),
    in_specs=[pl.BlockSpec((BN, 128), lambda i: (i, 0)),
              pl.BlockSpec((128, 128), lambda i: (0, 0)),
              pl.BlockSpec((128, 16), lambda i: (0, 0)),
              pl.BlockSpec((1, 16), lambda i: (0, 0))],
    out_specs=[pl.BlockSpec((BN, 128), lambda i: (i, 0)),
               pl.BlockSpec((BN, 16), lambda i: (i, 0))],
    out_shape=[jax.ShapeDtypeStruct((N, 128), jnp.float32),
               jax.ShapeDtypeStruct((N, 16), jnp.float32)],
)


def _mid_body(acc_ref, cnt_ref, o1_ref, b8_ref, s16_ref, h_ref):
    acc = acc_ref[0] + acc_ref[1]
    cnt = cnt_ref[0] + cnt_ref[1]
    cntw = jnp.dot(cnt, b8_ref[...], preferred_element_type=jnp.float32)
    inv = 1.0 / jnp.maximum(cntw, 1.0)
    hh = o1_ref[...] + jnp.dot(acc * inv, s16_ref[...],
                               preferred_element_type=jnp.float32)
    h_ref[...] = jnp.maximum(hh, 0.0)


_mid_call = pl.pallas_call(
    _mid_body,
    grid=(N // BN,),
    in_specs=[pl.BlockSpec((2, BN, 128), lambda i: (0, i, 0)),
              pl.BlockSpec((2, BN, 8), lambda i: (0, i, 0)),
              pl.BlockSpec((BN, 16), lambda i: (i, 0)),
              pl.BlockSpec((8, 128), lambda i: (0, 0)),
              pl.BlockSpec((128, 16), lambda i: (0, 0))],
    out_specs=pl.BlockSpec((BN, 16), lambda i: (i, 0)),
    out_shape=jax.ShapeDtypeStruct((N, 16), jnp.float32),
)


def _fin_body(acc_ref, cnt_ref, h_ref, r2_ref, b2_ref, b8_ref, w_ref,
              out_ref):
    acc = acc_ref[0] + acc_ref[1]
    cnt = cnt_ref[0] + cnt_ref[1]
    cntw = jnp.dot(cnt, b8_ref[...], preferred_element_type=jnp.float32)
    inv = 1.0 / jnp.maximum(cntw, 1.0)
    out_ref[...] = (jnp.dot(h_ref[...], r2_ref[...],
                            preferred_element_type=jnp.float32) + b2_ref[...]
                    + jnp.dot(acc * inv, w_ref[...],
                              preferred_element_type=jnp.float32))


_fin_call = pl.pallas_call(
    _fin_body,
    grid=(N // BN,),
    in_specs=[pl.BlockSpec((2, BN, 128), lambda i: (0, i, 0)),
              pl.BlockSpec((2, BN, 8), lambda i: (0, i, 0)),
              pl.BlockSpec((BN, 16), lambda i: (i, 0)),
              pl.BlockSpec((16, 128), lambda i: (0, 0)),
              pl.BlockSpec((1, 128), lambda i: (0, 0)),
              pl.BlockSpec((8, 128), lambda i: (0, 0)),
              pl.BlockSpec((128, 128), lambda i: (0, 0))],
    out_specs=pl.BlockSpec((BN, 128), lambda i: (i, 0)),
    out_shape=jax.ShapeDtypeStruct((N, 128), jnp.float32),
)


def kernel(x, edge_index, edge_type, W1, root1, b1, W2, root2, b2):
    src2 = edge_index[0].reshape(EB_ROWS, BT)
    dst2 = edge_index[1].reshape(EB_ROWS, BT)
    et2 = edge_type.reshape(EB_ROWS, BT)
    gidx2, sidx2, srcp2 = _idx_call(src2, dst2, et2)

    W1cat = W1.transpose(1, 0, 2).reshape(128, 128)
    xproj, o1root = _prep_call(x, W1cat, root1, b1.reshape(1, 16))

    acc1, cnt = _sc_pass1(gidx2, sidx2, xproj.reshape(N * R, 16))

    B8 = jnp.repeat(jnp.eye(R, dtype=jnp.float32), 16, axis=1)
    S16 = jnp.tile(jnp.eye(16, dtype=jnp.float32), (R, 1))
    cntr = cnt.reshape(2, NP_, R)
    h = _mid_call(acc1.reshape(2, NP_, 128), cntr, o1root, B8, S16)

    acc2 = _sc_pass2(srcp2, sidx2, h)
    out = _fin_call(acc2.reshape(2, NP_, 128), cntr, h, root2,
                    b2.reshape(1, 128), B8, W2.reshape(128, 128))
    return out


# speedup vs baseline: 2.1726x; 1.0211x over previous
import functools

import jax
import jax.numpy as jnp
from jax import lax
from jax.experimental import pallas as pl
from jax.experimental.pallas import tpu as pltpu
from jax.experimental.pallas import tpu_sc as plsc

N = 10000
E = 320000
R = 8
NP_ = 10240
NR = NP_ * R
NTILES = 32
EP = 327680
BT = 128
NB = EP // (NTILES * BT)
ROWS_PT = NR // 16
ZB = 128

_mesh = plsc.VectorSubcoreMesh(core_axis_name="c", subcore_axis_name="s")
_sc_params = pltpu.CompilerParams(use_tc_tiling_on_sc=False)


NBUF = 10
PF = 6


def _edge_sweep(tbl_hbm, gidx_v, sidx_v, rows_v, gsem, ssem, acc_s,
                cnt_fire=None):
    def g_start(j, b):
        pltpu.async_copy(tbl_hbm.at[gidx_v.at[j]], rows_v.at[b], gsem.at[b])

    def g_wait(b):
        pltpu.make_async_copy(tbl_hbm.at[gidx_v.at[0]], rows_v.at[b],
                              gsem.at[b]).wait()

    def s_start(j, b):
        pltpu.async_copy(rows_v.at[b], acc_s.at[sidx_v.at[j]], ssem.at[b],
                         add=True)

    def s_wait(b):
        pltpu.make_async_copy(rows_v.at[b], acc_s.at[sidx_v.at[0]],
                              ssem.at[b]).wait()

    def step(j, b, prefetch, wait_prev_scatter):
        g_wait(b)
        s_start(j, b)
        if cnt_fire is not None:
            cnt_fire(j)
        if prefetch:
            bp = (b + PF) % NBUF
            if wait_prev_scatter:
                s_wait(bp)
            g_start(j + PF, bp)

    for b in range(PF):
        g_start(b, b)
    for j in range(NBUF):
        step(j, j, prefetch=True, wait_prev_scatter=(j + PF >= NBUF))

    @pl.loop(NBUF, NB - NBUF, step=NBUF)
    def _(j0):
        for b in range(NBUF):
            step(j0 + b, b, prefetch=True, wait_prev_scatter=True)

    for j in range(NB - NBUF, NB):
        step(j, j % NBUF, prefetch=(j + PF < NB), wait_prev_scatter=True)
    for b in range(NBUF):
        s_wait(b)


def _sc_pass1_body(gidx_hbm, sidx_hbm, tbl_hbm, acc_out, cnt_out,
                   gidx_v, sidx_v, rows_v, ones_v, zbuf, zcnt, acc_s, cnt_s,
                   gsem, ssem, csem):
    cid = lax.axis_index("c")
    sid = lax.axis_index("s")
    tid = cid * 16 + sid

    @pl.loop(0, ZB)
    def _(i):
        zbuf[i] = jnp.zeros((16,), jnp.float32)

    @pl.loop(0, ZB, step=16)
    def _(i):
        zcnt[pl.ds(i, 16)] = jnp.zeros((16,), jnp.float32)

    for i in range(BT // 16):
        ones_v[pl.ds(i * 16, 16)] = jnp.ones((16,), jnp.float32)

    row0 = sid * ROWS_PT
    for k in range(ROWS_PT // ZB):
        pltpu.async_copy(zbuf, acc_s.at[pl.ds(row0 + k * ZB, ZB)], csem)
        pltpu.async_copy(zcnt, cnt_s.at[pl.ds(row0 + k * ZB, ZB)], csem)
    pltpu.async_copy(gidx_hbm.at[pl.ds(tid * NB, NB)], gidx_v, gsem.at[0])
    pltpu.async_copy(sidx_hbm.at[pl.ds(tid * NB, NB)], sidx_v, gsem.at[1])
    for k in range(ROWS_PT // ZB):
        pltpu.make_async_copy(zbuf, acc_s.at[pl.ds(row0, ZB)], csem).wait()
        pltpu.make_async_copy(zcnt, cnt_s.at[pl.ds(row0, ZB)], csem).wait()
    pltpu.make_async_copy(gidx_hbm.at[pl.ds(0, NB)], gidx_v, gsem.at[0]).wait()
    pltpu.make_async_copy(sidx_hbm.at[pl.ds(0, NB)], sidx_v, gsem.at[1]).wait()
    plsc.subcore_barrier()

    def cnt_fire(j):
        pltpu.async_copy(ones_v, cnt_s.at[sidx_v.at[j]], csem, add=True)

    _edge_sweep(tbl_hbm, gidx_v, sidx_v, rows_v, gsem, ssem, acc_s, cnt_fire)

    @pl.loop(0, NB)
    def _(_j):
        pltpu.make_async_copy(ones_v, cnt_s.at[sidx_v.at[0]], csem).wait()

    plsc.subcore_barrier()
    pltpu.sync_copy(acc_s.at[pl.ds(row0, ROWS_PT)],
                    acc_out.at[cid, pl.ds(row0, ROWS_PT)])
    pltpu.sync_copy(cnt_s.at[pl.ds(row0, ROWS_PT)],
                    cnt_out.at[cid, pl.ds(row0, ROWS_PT)])


_sc_pass1 = functools.partial(
    pl.kernel,
    out_type=[jax.ShapeDtypeStruct((2, NR, 16), jnp.float32),
              jax.ShapeDtypeStruct((2, NR), jnp.float32)],
    mesh=_mesh,
    scratch_types=[
        pltpu.VMEM((NB, BT), jnp.int32),
        pltpu.VMEM((NB, BT), jnp.int32),
        pltpu.VMEM((NBUF, BT, 16), jnp.float32),
        pltpu.VMEM((BT,), jnp.float32),
        pltpu.VMEM((ZB, 16), jnp.float32),
        pltpu.VMEM((ZB,), jnp.float32),
        pltpu.VMEM_SHARED((NR, 16), jnp.float32),
        pltpu.VMEM_SHARED((NR,), jnp.float32),
        pltpu.SemaphoreType.DMA((NBUF,)),
        pltpu.SemaphoreType.DMA((NBUF,)),
        pltpu.SemaphoreType.DMA,
    ],
    compiler_params=_sc_params,
)(_sc_pass1_body)


def _sc_pass2_body(gidx_hbm, sidx_hbm, tbl_hbm, acc_out,
                   gidx_v, sidx_v, rows_v, zbuf, acc_s, gsem, ssem):
    cid = lax.axis_index("c")
    sid = lax.axis_index("s")
    tid = cid * 16 + sid

    @pl.loop(0, ZB)
    def _(i):
        zbuf[i] = jnp.zeros((16,), jnp.float32)

    row0 = sid * ROWS_PT
    for k in range(ROWS_PT // ZB):
        pltpu.async_copy(zbuf, acc_s.at[pl.ds(row0 + k * ZB, ZB)], ssem.at[0])
    pltpu.async_copy(gidx_hbm.at[pl.ds(tid * NB, NB)], gidx_v, gsem.at[0])
    pltpu.async_copy(sidx_hbm.at[pl.ds(tid * NB, NB)], sidx_v, gsem.at[1])
    for k in range(ROWS_PT // ZB):
        pltpu.make_async_copy(zbuf, acc_s.at[pl.ds(row0, ZB)],
                              ssem.at[0]).wait()
    pltpu.make_async_copy(gidx_hbm.at[pl.ds(0, NB)], gidx_v, gsem.at[0]).wait()
    pltpu.make_async_copy(sidx_hbm.at[pl.ds(0, NB)], sidx_v, gsem.at[1]).wait()
    plsc.subcore_barrier()

    _edge_sweep(tbl_hbm, gidx_v, sidx_v, rows_v, gsem, ssem, acc_s)

    plsc.subcore_barrier()
    pltpu.sync_copy(acc_s.at[pl.ds(row0, ROWS_PT)],
                    acc_out.at[cid, pl.ds(row0, ROWS_PT)])


_sc_pass2 = functools.partial(
    pl.kernel,
    out_type=jax.ShapeDtypeStruct((2, NR, 16), jnp.float32),
    mesh=_mesh,
    scratch_types=[
        pltpu.VMEM((NB, BT), jnp.int32),
        pltpu.VMEM((NB, BT), jnp.int32),
        pltpu.VMEM((NBUF, BT, 16), jnp.float32),
        pltpu.VMEM((ZB, 16), jnp.float32),
        pltpu.VMEM_SHARED((NR, 16), jnp.float32),
        pltpu.SemaphoreType.DMA((NBUF,)),
        pltpu.SemaphoreType.DMA((NBUF,)),
    ],
    compiler_params=_sc_params,
)(_sc_pass2_body)


EB_ROWS = E // BT
PAD_ROWS = EP // BT - EB_ROWS


def _idx_body(src_ref, dst_ref, et_ref, g_ref, s_ref, sp_ref):
    et = et_ref[...]
    g_ref[0:EB_ROWS] = src_ref[...] * R + et
    s_ref[0:EB_ROWS] = dst_ref[...] * R + et
    sp_ref[0:EB_ROWS] = src_ref[...]
    pidx = (lax.broadcasted_iota(jnp.int32, (PAD_ROWS, BT), 0) * BT
            + lax.broadcasted_iota(jnp.int32, (PAD_ROWS, BT), 1))
    spread = pidx % (NR - N * R)
    g_ref[EB_ROWS:] = spread
    s_ref[EB_ROWS:] = N * R + spread
    sp_ref[EB_ROWS:] = spread


_idx_call = pl.pallas_call(
    _idx_body,
    out_shape=[jax.ShapeDtypeStruct((EP // BT, BT), jnp.int32),
               jax.ShapeDtypeStruct((EP // BT, BT), jnp.int32),
               jax.ShapeDtypeStruct((EP // BT, BT), jnp.int32)],
)

BN = 2000


def _prep_body(x_ref, w_ref, r_ref, b_ref, xp_ref, o1_ref):
    xb = x_ref[...]
    xp_ref[...] = jnp.dot(xb, w_ref[...], preferred_element_type=jnp.float32)
    o1_ref[...] = jnp.dot(xb, r_ref[...],
                          preferred_element_type=jnp.float32) + b_ref[...]


_prep_call = pl.pallas_call(
    _prep_body,
    grid=(N // BN,),
    in_specs=[pl.BlockSpec((BN, 128), lambda i: (i, 0)),
              pl.BlockSpec((128, 128), lambda i: (0, 0)),
              pl.BlockSpec((128, 16), lambda i: (0, 0)),
              pl.BlockSpec((1, 16), lambda i: (0, 0))],
    out_specs=[pl.BlockSpec((BN, 128), lambda i: (i, 0)),
               pl.BlockSpec((BN, 16), lambda i: (i, 0))],
    out_shape=[jax.ShapeDtypeStruct((N, 128), jnp.float32),
               jax.ShapeDtypeStruct((N, 16), jnp.float32)],
)


def _mid_body(acc_ref, cnt_ref, o1_ref, b8_ref, s16_ref, h_ref):
    acc = acc_ref[0] + acc_ref[1]
    cnt = cnt_ref[0] + cnt_ref[1]
    cntw = jnp.dot(cnt, b8_ref[...], preferred_element_type=jnp.float32)
    inv = 1.0 / jnp.maximum(cntw, 1.0)
    hh = o1_ref[...] + jnp.dot(acc * inv, s16_ref[...],
                               preferred_element_type=jnp.float32)
    h_ref[...] = jnp.maximum(hh, 0.0)


_mid_call = pl.pallas_call(
    _mid_body,
    grid=(N // BN,),
    in_specs=[pl.BlockSpec((2, BN, 128), lambda i: (0, i, 0)),
              pl.BlockSpec((2, BN, 8), lambda i: (0, i, 0)),
              pl.BlockSpec((BN, 16), lambda i: (i, 0)),
              pl.BlockSpec((8, 128), lambda i: (0, 0)),
              pl.BlockSpec((128, 16), lambda i: (0, 0))],
    out_specs=pl.BlockSpec((BN, 16), lambda i: (i, 0)),
    out_shape=jax.ShapeDtypeStruct((N, 16), jnp.float32),
)


def _fin_body(acc_ref, cnt_ref, h_ref, r2_ref, b2_ref, b8_ref, w_ref,
              out_ref):
    acc = acc_ref[0] + acc_ref[1]
    cnt = cnt_ref[0] + cnt_ref[1]
    cntw = jnp.dot(cnt, b8_ref[...], preferred_element_type=jnp.float32)
    inv = 1.0 / jnp.maximum(cntw, 1.0)
    out_ref[...] = (jnp.dot(h_ref[...], r2_ref[...],
                            preferred_element_type=jnp.float32) + b2_ref[...]
                    + jnp.dot(acc * inv, w_ref[...],
                              preferred_element_type=jnp.float32))


_fin_call = pl.pallas_call(
    _fin_body,
    grid=(N // BN,),
    in_specs=[pl.BlockSpec((2, BN, 128), lambda i: (0, i, 0)),
              pl.BlockSpec((2, BN, 8), lambda i: (0, i, 0)),
              pl.BlockSpec((BN, 16), lambda i: (i, 0)),
              pl.BlockSpec((16, 128), lambda i: (0, 0)),
              pl.BlockSpec((1, 128), lambda i: (0, 0)),
              pl.BlockSpec((8, 128), lambda i: (0, 0)),
              pl.BlockSpec((128, 128), lambda i: (0, 0))],
    out_specs=pl.BlockSpec((BN, 128), lambda i: (i, 0)),
    out_shape=jax.ShapeDtypeStruct((N, 128), jnp.float32),
)


def kernel(x, edge_index, edge_type, W1, root1, b1, W2, root2, b2):
    src2 = edge_index[0].reshape(EB_ROWS, BT)
    dst2 = edge_index[1].reshape(EB_ROWS, BT)
    et2 = edge_type.reshape(EB_ROWS, BT)
    gidx2, sidx2, srcp2 = _idx_call(src2, dst2, et2)

    W1cat = W1.transpose(1, 0, 2).reshape(128, 128)
    xproj, o1root = _prep_call(x, W1cat, root1, b1.reshape(1, 16))

    acc1, cnt = _sc_pass1(gidx2, sidx2, xproj.reshape(N * R, 16))

    B8 = jnp.repeat(jnp.eye(R, dtype=jnp.float32), 16, axis=1)
    S16 = jnp.tile(jnp.eye(16, dtype=jnp.float32), (R, 1))
    cntr = cnt.reshape(2, NP_, R)
    h = _mid_call(acc1.reshape(2, NP_, 128), cntr, o1root, B8, S16)

    acc2 = _sc_pass2(srcp2, sidx2, h)
    out = _fin_call(acc2.reshape(2, NP_, 128), cntr, h, root2,
                    b2.reshape(1, 128), B8, W2.reshape(128, 128))
    return out


# speedup vs baseline: 2.2052x; 1.0150x over previous
import functools

import jax
import jax.numpy as jnp
from jax import lax
from jax.experimental import pallas as pl
from jax.experimental.pallas import tpu as pltpu
from jax.experimental.pallas import tpu_sc as plsc

N = 10000
E = 320000
R = 8
NP_ = 10240
NR = NP_ * R
NTILES = 32
EP = 327680
BT = 128
NB = EP // (NTILES * BT)
ROWS_PT = NR // 16
ZB = 128

_mesh = plsc.VectorSubcoreMesh(core_axis_name="c", subcore_axis_name="s")
_sc_params = pltpu.CompilerParams(use_tc_tiling_on_sc=False)


NBUF = 10
PF = 7


def _edge_sweep(tbl_hbm, gidx_v, sidx_v, rows_v, gsem, ssem, acc_s,
                cnt_fire=None):
    def g_start(j, b):
        pltpu.async_copy(tbl_hbm.at[gidx_v.at[j]], rows_v.at[b], gsem.at[b])

    def g_wait(b):
        pltpu.make_async_copy(tbl_hbm.at[gidx_v.at[0]], rows_v.at[b],
                              gsem.at[b]).wait()

    def s_start(j, b):
        pltpu.async_copy(rows_v.at[b], acc_s.at[sidx_v.at[j]], ssem.at[b],
                         add=True)

    def s_wait(b):
        pltpu.make_async_copy(rows_v.at[b], acc_s.at[sidx_v.at[0]],
                              ssem.at[b]).wait()

    def step(j, b, prefetch, wait_prev_scatter):
        g_wait(b)
        s_start(j, b)
        if cnt_fire is not None:
            cnt_fire(j)
        if prefetch:
            bp = (b + PF) % NBUF
            if wait_prev_scatter:
                s_wait(bp)
            g_start(j + PF, bp)

    for b in range(PF):
        g_start(b, b)
    for j in range(NBUF):
        step(j, j, prefetch=True, wait_prev_scatter=(j + PF >= NBUF))

    @pl.loop(NBUF, NB - NBUF, step=NBUF)
    def _(j0):
        for b in range(NBUF):
            step(j0 + b, b, prefetch=True, wait_prev_scatter=True)

    for j in range(NB - NBUF, NB):
        step(j, j % NBUF, prefetch=(j + PF < NB), wait_prev_scatter=True)
    for b in range(NBUF):
        s_wait(b)


def _sc_pass1_body(gidx_hbm, sidx_hbm, tbl_hbm, acc_out, cnt_out,
                   gidx_v, sidx_v, rows_v, ones_v, zbuf, zcnt, acc_s, cnt_s,
                   gsem, ssem, csem):
    cid = lax.axis_index("c")
    sid = lax.axis_index("s")
    tid = cid * 16 + sid

    @pl.loop(0, ZB)
    def _(i):
        zbuf[i] = jnp.zeros((16,), jnp.float32)

    @pl.loop(0, ZB, step=16)
    def _(i):
        zcnt[pl.ds(i, 16)] = jnp.zeros((16,), jnp.float32)

    for i in range(BT // 16):
        ones_v[pl.ds(i * 16, 16)] = jnp.ones((16,), jnp.float32)

    row0 = sid * ROWS_PT
    for k in range(ROWS_PT // ZB):
        pltpu.async_copy(zbuf, acc_s.at[pl.ds(row0 + k * ZB, ZB)], csem)
        pltpu.async_copy(zcnt, cnt_s.at[pl.ds(row0 + k * ZB, ZB)], csem)
    pltpu.async_copy(gidx_hbm.at[pl.ds(tid * NB, NB)], gidx_v, gsem.at[0])
    pltpu.async_copy(sidx_hbm.at[pl.ds(tid * NB, NB)], sidx_v, gsem.at[1])
    for k in range(ROWS_PT // ZB):
        pltpu.make_async_copy(zbuf, acc_s.at[pl.ds(row0, ZB)], csem).wait()
        pltpu.make_async_copy(zcnt, cnt_s.at[pl.ds(row0, ZB)], csem).wait()
    pltpu.make_async_copy(gidx_hbm.at[pl.ds(0, NB)], gidx_v, gsem.at[0]).wait()
    pltpu.make_async_copy(sidx_hbm.at[pl.ds(0, NB)], sidx_v, gsem.at[1]).wait()
    plsc.subcore_barrier()

    def cnt_fire(j):
        pltpu.async_copy(ones_v, cnt_s.at[sidx_v.at[j]], csem, add=True)

    _edge_sweep(tbl_hbm, gidx_v, sidx_v, rows_v, gsem, ssem, acc_s, cnt_fire)

    @pl.loop(0, NB)
    def _(_j):
        pltpu.make_async_copy(ones_v, cnt_s.at[sidx_v.at[0]], csem).wait()

    plsc.subcore_barrier()
    pltpu.sync_copy(acc_s.at[pl.ds(row0, ROWS_PT)],
                    acc_out.at[cid, pl.ds(row0, ROWS_PT)])
    pltpu.sync_copy(cnt_s.at[pl.ds(row0, ROWS_PT)],
                    cnt_out.at[cid, pl.ds(row0, ROWS_PT)])


_sc_pass1 = functools.partial(
    pl.kernel,
    out_type=[jax.ShapeDtypeStruct((2, NR, 16), jnp.float32),
              jax.ShapeDtypeStruct((2, NR), jnp.float32)],
    mesh=_mesh,
    scratch_types=[
        pltpu.VMEM((NB, BT), jnp.int32),
        pltpu.VMEM((NB, BT), jnp.int32),
        pltpu.VMEM((NBUF, BT, 16), jnp.float32),
        pltpu.VMEM((BT,), jnp.float32),
        pltpu.VMEM((ZB, 16), jnp.float32),
        pltpu.VMEM((ZB,), jnp.float32),
        pltpu.VMEM_SHARED((NR, 16), jnp.float32),
        pltpu.VMEM_SHARED((NR,), jnp.float32),
        pltpu.SemaphoreType.DMA((NBUF,)),
        pltpu.SemaphoreType.DMA((NBUF,)),
        pltpu.SemaphoreType.DMA,
    ],
    compiler_params=_sc_params,
)(_sc_pass1_body)


def _sc_pass2_body(gidx_hbm, sidx_hbm, tbl_hbm, acc_out,
                   gidx_v, sidx_v, rows_v, zbuf, acc_s, gsem, ssem):
    cid = lax.axis_index("c")
    sid = lax.axis_index("s")
    tid = cid * 16 + sid

    @pl.loop(0, ZB)
    def _(i):
        zbuf[i] = jnp.zeros((16,), jnp.float32)

    row0 = sid * ROWS_PT
    for k in range(ROWS_PT // ZB):
        pltpu.async_copy(zbuf, acc_s.at[pl.ds(row0 + k * ZB, ZB)], ssem.at[0])
    pltpu.async_copy(gidx_hbm.at[pl.ds(tid * NB, NB)], gidx_v, gsem.at[0])
    pltpu.async_copy(sidx_hbm.at[pl.ds(tid * NB, NB)], sidx_v, gsem.at[1])
    for k in range(ROWS_PT // ZB):
        pltpu.make_async_copy(zbuf, acc_s.at[pl.ds(row0, ZB)],
                              ssem.at[0]).wait()
    pltpu.make_async_copy(gidx_hbm.at[pl.ds(0, NB)], gidx_v, gsem.at[0]).wait()
    pltpu.make_async_copy(sidx_hbm.at[pl.ds(0, NB)], sidx_v, gsem.at[1]).wait()
    plsc.subcore_barrier()

    _edge_sweep(tbl_hbm, gidx_v, sidx_v, rows_v, gsem, ssem, acc_s)

    plsc.subcore_barrier()
    pltpu.sync_copy(acc_s.at[pl.ds(row0, ROWS_PT)],
                    acc_out.at[cid, pl.ds(row0, ROWS_PT)])


_sc_pass2 = functools.partial(
    pl.kernel,
    out_type=jax.ShapeDtypeStruct((2, NR, 16), jnp.float32),
    mesh=_mesh,
    scratch_types=[
        pltpu.VMEM((NB, BT), jnp.int32),
        pltpu.VMEM((NB, BT), jnp.int32),
        pltpu.VMEM((NBUF, BT, 16), jnp.float32),
        pltpu.VMEM((ZB, 16), jnp.float32),
        pltpu.VMEM_SHARED((NR, 16), jnp.float32),
        pltpu.SemaphoreType.DMA((NBUF,)),
        pltpu.SemaphoreType.DMA((NBUF,)),
    ],
    compiler_params=_sc_params,
)(_sc_pass2_body)


EB_ROWS = E // BT
PAD_ROWS = EP // BT - EB_ROWS


def _idx_body(src_ref, dst_ref, et_ref, g_ref, s_ref, sp_ref):
    et = et_ref[...]
    g_ref[0:EB_ROWS] = src_ref[...] * R + et
    s_ref[0:EB_ROWS] = dst_ref[...] * R + et
    sp_ref[0:EB_ROWS] = src_ref[...]
    pidx = (lax.broadcasted_iota(jnp.int32, (PAD_ROWS, BT), 0) * BT
            + lax.broadcasted_iota(jnp.int32, (PAD_ROWS, BT), 1))
    spread = pidx % (NR - N * R)
    g_ref[EB_ROWS:] = spread
    s_ref[EB_ROWS:] = N * R + spread
    sp_ref[EB_ROWS:] = spread


_idx_call = pl.pallas_call(
    _idx_body,
    out_shape=[jax.ShapeDtypeStruct((EP // BT, BT), jnp.int32),
               jax.ShapeDtypeStruct((EP // BT, BT), jnp.int32),
               jax.ShapeDtypeStruct((EP // BT, BT), jnp.int32)],
)

BN = 2000


def _prep_body(x_ref, w_ref, r_ref, b_ref, xp_ref, o1_ref):
    xb = x_ref[...]
    xp_ref[...] = jnp.dot(xb, w_ref[...], preferred_element_type=jnp.float32)
    o1_ref[...] = jnp.dot(xb, r_ref[...],
                          preferred_element_type=jnp.float32) + b_ref[...]


_prep_call = pl.pallas_call(
    _prep_body,
    grid=(N // BN,),
    in_specs=[pl.BlockSpec((BN, 128), lambda i: (i, 0)),
              pl.BlockSpec((128, 128), lambda i: (0, 0)),
              pl.BlockSpec((128, 16), lambda i: (0, 0)),
              pl.BlockSpec((1, 16), lambda i: (0, 0))],
    out_specs=[pl.BlockSpec((BN, 128), lambda i: (i, 0)),
               pl.BlockSpec((BN, 16), lambda i: (i, 0))],
    out_shape=[jax.ShapeDtypeStruct((N, 128), jnp.float32),
               jax.ShapeDtypeStruct((N, 16), jnp.float32)],
)


def _mid_body(acc_ref, cnt_ref, o1_ref, b8_ref, s16_ref, h_ref):
    acc = acc_ref[0] + acc_ref[1]
    cnt = cnt_ref[0] + cnt_ref[1]
    cntw = jnp.dot(cnt, b8_ref[...], preferred_element_type=jnp.float32)
    inv = 1.0 / jnp.maximum(cntw, 1.0)
    hh = o1_ref[...] + jnp.dot(acc * inv, s16_ref[...],
                               preferred_element_type=jnp.float32)
    h_ref[...] = jnp.maximum(hh, 0.0)


_mid_call = pl.pallas_call(
    _mid_body,
    grid=(N // BN,),
    in_specs=[pl.BlockSpec((2, BN, 128), lambda i: (0, i, 0)),
              pl.BlockSpec((2, BN, 8), lambda i: (0, i, 0)),
              pl.BlockSpec((BN, 16), lambda i: (i, 0)),
              pl.BlockSpec((8, 128), lambda i: (0, 0)),
              pl.BlockSpec((128, 16), lambda i: (0, 0))],
    out_specs=pl.BlockSpec((BN, 16), lambda i: (i, 0)),
    out_shape=jax.ShapeDtypeStruct((N, 16), jnp.float32),
)


def _fin_body(acc_ref, cnt_ref, h_ref, r2_ref, b2_ref, b8_ref, w_ref,
              out_ref):
    acc = acc_ref[0] + acc_ref[1]
    cnt = cnt_ref[0] + cnt_ref[1]
    cntw = jnp.dot(cnt, b8_ref[...], preferred_element_type=jnp.float32)
    inv = 1.0 / jnp.maximum(cntw, 1.0)
    out_ref[...] = (jnp.dot(h_ref[...], r2_ref[...],
                            preferred_element_type=jnp.float32) + b2_ref[...]
                    + jnp.dot(acc * inv, w_ref[...],
                              preferred_element_type=jnp.float32))


_fin_call = pl.pallas_call(
    _fin_body,
    grid=(N // BN,),
    in_specs=[pl.BlockSpec((2, BN, 128), lambda i: (0, i, 0)),
              pl.BlockSpec((2, BN, 8), lambda i: (0, i, 0)),
              pl.BlockSpec((BN, 16), lambda i: (i, 0)),
              pl.BlockSpec((16, 128), lambda i: (0, 0)),
              pl.BlockSpec((1, 128), lambda i: (0, 0)),
              pl.BlockSpec((8, 128), lambda i: (0, 0)),
              pl.BlockSpec((128, 128), lambda i: (0, 0))],
    out_specs=pl.BlockSpec((BN, 128), lambda i: (i, 0)),
    out_shape=jax.ShapeDtypeStruct((N, 128), jnp.float32),
)


def kernel(x, edge_index, edge_type, W1, root1, b1, W2, root2, b2):
    src2 = edge_index[0].reshape(EB_ROWS, BT)
    dst2 = edge_index[1].reshape(EB_ROWS, BT)
    et2 = edge_type.reshape(EB_ROWS, BT)
    gidx2, sidx2, srcp2 = _idx_call(src2, dst2, et2)

    W1cat = W1.transpose(1, 0, 2).reshape(128, 128)
    xproj, o1root = _prep_call(x, W1cat, root1, b1.reshape(1, 16))

    acc1, cnt = _sc_pass1(gidx2, sidx2, xproj.reshape(N * R, 16))

    B8 = jnp.repeat(jnp.eye(R, dtype=jnp.float32), 16, axis=1)
    S16 = jnp.tile(jnp.eye(16, dtype=jnp.float32), (R, 1))
    cntr = cnt.reshape(2, NP_, R)
    h = _mid_call(acc1.reshape(2, NP_, 128), cntr, o1root, B8, S16)

    acc2 = _sc_pass2(srcp2, sidx2, h)
    out = _fin_call(acc2.reshape(2, NP_, 128), cntr, h, root2,
                    b2.reshape(1, 128), B8, W2.reshape(128, 128))
    return out


# speedup vs baseline: 2.2235x; 1.0083x over previous
import functools

import jax
import jax.numpy as jnp
from jax import lax
from jax.experimental import pallas as pl
from jax.experimental.pallas import tpu as pltpu
from jax.experimental.pallas import tpu_sc as plsc

N = 10000
E = 320000
R = 8
NP_ = 10240
NR = NP_ * R
NTILES = 32
EP = 327680
BT = 128
NB = EP // (NTILES * BT)
ROWS_PT = NR // 16
ZB = 128

_mesh = plsc.VectorSubcoreMesh(core_axis_name="c", subcore_axis_name="s")
_sc_params = pltpu.CompilerParams(use_tc_tiling_on_sc=False)


NBUF = 10
PF = 8


def _edge_sweep(tbl_hbm, gidx_v, sidx_v, rows_v, gsem, ssem, acc_s,
                cnt_fire=None):
    def g_start(j, b):
        pltpu.async_copy(tbl_hbm.at[gidx_v.at[j]], rows_v.at[b], gsem.at[b])

    def g_wait(b):
        pltpu.make_async_copy(tbl_hbm.at[gidx_v.at[0]], rows_v.at[b],
                              gsem.at[b]).wait()

    def s_start(j, b):
        pltpu.async_copy(rows_v.at[b], acc_s.at[sidx_v.at[j]], ssem.at[b],
                         add=True)

    def s_wait(b):
        pltpu.make_async_copy(rows_v.at[b], acc_s.at[sidx_v.at[0]],
                              ssem.at[b]).wait()

    def step(j, b, prefetch, wait_prev_scatter):
        g_wait(b)
        s_start(j, b)
        if cnt_fire is not None:
            cnt_fire(j)
        if prefetch:
            bp = (b + PF) % NBUF
            if wait_prev_scatter:
                s_wait(bp)
            g_start(j + PF, bp)

    for b in range(PF):
        g_start(b, b)
    for j in range(NBUF):
        step(j, j, prefetch=True, wait_prev_scatter=(j + PF >= NBUF))

    @pl.loop(NBUF, NB - NBUF, step=NBUF)
    def _(j0):
        for b in range(NBUF):
            step(j0 + b, b, prefetch=True, wait_prev_scatter=True)

    for j in range(NB - NBUF, NB):
        step(j, j % NBUF, prefetch=(j + PF < NB), wait_prev_scatter=True)
    for b in range(NBUF):
        s_wait(b)


def _sc_pass1_body(gidx_hbm, sidx_hbm, tbl_hbm, acc_out, cnt_out,
                   gidx_v, sidx_v, rows_v, ones_v, zbuf, zcnt, acc_s, cnt_s,
                   gsem, ssem, csem):
    cid = lax.axis_index("c")
    sid = lax.axis_index("s")
    tid = cid * 16 + sid

    @pl.loop(0, ZB)
    def _(i):
        zbuf[i] = jnp.zeros((16,), jnp.float32)

    @pl.loop(0, ZB, step=16)
    def _(i):
        zcnt[pl.ds(i, 16)] = jnp.zeros((16,), jnp.float32)

    for i in range(BT // 16):
        ones_v[pl.ds(i * 16, 16)] = jnp.ones((16,), jnp.float32)

    row0 = sid * ROWS_PT
    for k in range(ROWS_PT // ZB):
        pltpu.async_copy(zbuf, acc_s.at[pl.ds(row0 + k * ZB, ZB)], csem)
        pltpu.async_copy(zcnt, cnt_s.at[pl.ds(row0 + k * ZB, ZB)], csem)
    pltpu.async_copy(gidx_hbm.at[pl.ds(tid * NB, NB)], gidx_v, gsem.at[0])
    pltpu.async_copy(sidx_hbm.at[pl.ds(tid * NB, NB)], sidx_v, gsem.at[1])
    for k in range(ROWS_PT // ZB):
        pltpu.make_async_copy(zbuf, acc_s.at[pl.ds(row0, ZB)], csem).wait()
        pltpu.make_async_copy(zcnt, cnt_s.at[pl.ds(row0, ZB)], csem).wait()
    pltpu.make_async_copy(gidx_hbm.at[pl.ds(0, NB)], gidx_v, gsem.at[0]).wait()
    pltpu.make_async_copy(sidx_hbm.at[pl.ds(0, NB)], sidx_v, gsem.at[1]).wait()
    plsc.subcore_barrier()

    def cnt_fire(j):
        pltpu.async_copy(ones_v, cnt_s.at[sidx_v.at[j]], csem, add=True)

    _edge_sweep(tbl_hbm, gidx_v, sidx_v, rows_v, gsem, ssem, acc_s, cnt_fire)

    @pl.loop(0, NB)
    def _(_j):
        pltpu.make_async_copy(ones_v, cnt_s.at[sidx_v.at[0]], csem).wait()

    plsc.subcore_barrier()
    pltpu.sync_copy(acc_s.at[pl.ds(row0, ROWS_PT)],
                    acc_out.at[cid, pl.ds(row0, ROWS_PT)])
    pltpu.sync_copy(cnt_s.at[pl.ds(row0, ROWS_PT)],
                    cnt_out.at[cid, pl.ds(row0, ROWS_PT)])


_sc_pass1 = functools.partial(
    pl.kernel,
    out_type=[jax.ShapeDtypeStruct((2, NR, 16), jnp.float32),
              jax.ShapeDtypeStruct((2, NR), jnp.float32)],
    mesh=_mesh,
    scratch_types=[
        pltpu.VMEM((NB, BT), jnp.int32),
        pltpu.VMEM((NB, BT), jnp.int32),
        pltpu.VMEM((NBUF, BT, 16), jnp.float32),
        pltpu.VMEM((BT,), jnp.float32),
        pltpu.VMEM((ZB, 16), jnp.float32),
        pltpu.VMEM((ZB,), jnp.float32),
        pltpu.VMEM_SHARED((NR, 16), jnp.float32),
        pltpu.VMEM_SHARED((NR,), jnp.float32),
        pltpu.SemaphoreType.DMA((NBUF,)),
        pltpu.SemaphoreType.DMA((NBUF,)),
        pltpu.SemaphoreType.DMA,
    ],
    compiler_params=_sc_params,
)(_sc_pass1_body)


def _sc_pass2_body(gidx_hbm, sidx_hbm, tbl_hbm, acc_out,
                   gidx_v, sidx_v, rows_v, zbuf, acc_s, gsem, ssem):
    cid = lax.axis_index("c")
    sid = lax.axis_index("s")
    tid = cid * 16 + sid

    @pl.loop(0, ZB)
    def _(i):
        zbuf[i] = jnp.zeros((16,), jnp.float32)

    row0 = sid * ROWS_PT
    for k in range(ROWS_PT // ZB):
        pltpu.async_copy(zbuf, acc_s.at[pl.ds(row0 + k * ZB, ZB)], ssem.at[0])
    pltpu.async_copy(gidx_hbm.at[pl.ds(tid * NB, NB)], gidx_v, gsem.at[0])
    pltpu.async_copy(sidx_hbm.at[pl.ds(tid * NB, NB)], sidx_v, gsem.at[1])
    for k in range(ROWS_PT // ZB):
        pltpu.make_async_copy(zbuf, acc_s.at[pl.ds(row0, ZB)],
                              ssem.at[0]).wait()
    pltpu.make_async_copy(gidx_hbm.at[pl.ds(0, NB)], gidx_v, gsem.at[0]).wait()
    pltpu.make_async_copy(sidx_hbm.at[pl.ds(0, NB)], sidx_v, gsem.at[1]).wait()
    plsc.subcore_barrier()

    _edge_sweep(tbl_hbm, gidx_v, sidx_v, rows_v, gsem, ssem, acc_s)

    plsc.subcore_barrier()
    pltpu.sync_copy(acc_s.at[pl.ds(row0, ROWS_PT)],
                    acc_out.at[cid, pl.ds(row0, ROWS_PT)])


_sc_pass2 = functools.partial(
    pl.kernel,
    out_type=jax.ShapeDtypeStruct((2, NR, 16), jnp.float32),
    mesh=_mesh,
    scratch_types=[
        pltpu.VMEM((NB, BT), jnp.int32),
        pltpu.VMEM((NB, BT), jnp.int32),
        pltpu.VMEM((NBUF, BT, 16), jnp.float32),
        pltpu.VMEM((ZB, 16), jnp.float32),
        pltpu.VMEM_SHARED((NR, 16), jnp.float32),
        pltpu.SemaphoreType.DMA((NBUF,)),
        pltpu.SemaphoreType.DMA((NBUF,)),
    ],
    compiler_params=_sc_params,
)(_sc_pass2_body)


EB_ROWS = E // BT
PAD_ROWS = EP // BT - EB_ROWS


def _idx_body(src_ref, dst_ref, et_ref, g_ref, s_ref, sp_ref):
    et = et_ref[...]
    g_ref[0:EB_ROWS] = src_ref[...] * R + et
    s_ref[0:EB_ROWS] = dst_ref[...] * R + et
    sp_ref[0:EB_ROWS] = src_ref[...]
    pidx = (lax.broadcasted_iota(jnp.int32, (PAD_ROWS, BT), 0) * BT
            + lax.broadcasted_iota(jnp.int32, (PAD_ROWS, BT), 1))
    spread = pidx % (NR - N * R)
    g_ref[EB_ROWS:] = spread
    s_ref[EB_ROWS:] = N * R + spread
    sp_ref[EB_ROWS:] = spread


_idx_call = pl.pallas_call(
    _idx_body,
    out_shape=[jax.ShapeDtypeStruct((EP // BT, BT), jnp.int32),
               jax.ShapeDtypeStruct((EP // BT, BT), jnp.int32),
               jax.ShapeDtypeStruct((EP // BT, BT), jnp.int32)],
)

BN = 2000


def _prep_body(x_ref, w_ref, r_ref, b_ref, xp_ref, o1_ref):
    xb = x_ref[...]
    xp_ref[...] = jnp.dot(xb, w_ref[...], preferred_element_type=jnp.float32)
    o1_ref[...] = jnp.dot(xb, r_ref[...],
                          preferred_element_type=jnp.float32) + b_ref[...]


_prep_call = pl.pallas_call(
    _prep_body,
    grid=(N // BN,),
    in_specs=[pl.BlockSpec((BN, 128), lambda i: (i, 0)),
              pl.BlockSpec((128, 128), lambda i: (0, 0)),
              pl.BlockSpec((128, 16), lambda i: (0, 0)),
              pl.BlockSpec((1, 16), lambda i: (0, 0))],
    out_specs=[pl.BlockSpec((BN, 128), lambda i: (i, 0)),
               pl.BlockSpec((BN, 16), lambda i: (i, 0))],
    out_shape=[jax.ShapeDtypeStruct((N, 128), jnp.float32),
               jax.ShapeDtypeStruct((N, 16), jnp.float32)],
)


def _mid_body(acc_ref, cnt_ref, o1_ref, b8_ref, s16_ref, h_ref):
    acc = acc_ref[0] + acc_ref[1]
    cnt = cnt_ref[0] + cnt_ref[1]
    cntw = jnp.dot(cnt, b8_ref[...], preferred_element_type=jnp.float32)
    inv = 1.0 / jnp.maximum(cntw, 1.0)
    hh = o1_ref[...] + jnp.dot(acc * inv, s16_ref[...],
                               preferred_element_type=jnp.float32)
    h_ref[...] = jnp.maximum(hh, 0.0)


_mid_call = pl.pallas_call(
    _mid_body,
    grid=(N // BN,),
    in_specs=[pl.BlockSpec((2, BN, 128), lambda i: (0, i, 0)),
              pl.BlockSpec((2, BN, 8), lambda i: (0, i, 0)),
              pl.BlockSpec((BN, 16), lambda i: (i, 0)),
              pl.BlockSpec((8, 128), lambda i: (0, 0)),
              pl.BlockSpec((128, 16), lambda i: (0, 0))],
    out_specs=pl.BlockSpec((BN, 16), lambda i: (i, 0)),
    out_shape=jax.ShapeDtypeStruct((N, 16), jnp.float32),
)


def _fin_body(acc_ref, cnt_ref, h_ref, r2_ref, b2_ref, b8_ref, w_ref,
              out_ref):
    acc = acc_ref[0] + acc_ref[1]
    cnt = cnt_ref[0] + cnt_ref[1]
    cntw = jnp.dot(cnt, b8_ref[...], preferred_element_type=jnp.float32)
    inv = 1.0 / jnp.maximum(cntw, 1.0)
    out_ref[...] = (jnp.dot(h_ref[...], r2_ref[...],
                            preferred_element_type=jnp.float32) + b2_ref[...]
                    + jnp.dot(acc * inv, w_ref[...],
                              preferred_element_type=jnp.float32))


_fin_call = pl.pallas_call(
    _fin_body,
    grid=(N // BN,),
    in_specs=[pl.BlockSpec((2, BN, 128), lambda i: (0, i, 0)),
              pl.BlockSpec((2, BN, 8), lambda i: (0, i, 0)),
              pl.BlockSpec((BN, 16), lambda i: (i, 0)),
              pl.BlockSpec((16, 128), lambda i: (0, 0)),
              pl.BlockSpec((1, 128), lambda i: (0, 0)),
              pl.BlockSpec((8, 128), lambda i: (0, 0)),
              pl.BlockSpec((128, 128), lambda i: (0, 0))],
    out_specs=pl.BlockSpec((BN, 128), lambda i: (i, 0)),
    out_shape=jax.ShapeDtypeStruct((N, 128), jnp.float32),
)


def kernel(x, edge_index, edge_type, W1, root1, b1, W2, root2, b2):
    src2 = edge_index[0].reshape(EB_ROWS, BT)
    dst2 = edge_index[1].reshape(EB_ROWS, BT)
    et2 = edge_type.reshape(EB_ROWS, BT)
    gidx2, sidx2, srcp2 = _idx_call(src2, dst2, et2)

    W1cat = W1.transpose(1, 0, 2).reshape(128, 128)
    xproj, o1root = _prep_call(x, W1cat, root1, b1.reshape(1, 16))

    acc1, cnt = _sc_pass1(gidx2, sidx2, xproj.reshape(N * R, 16))

    B8 = jnp.repeat(jnp.eye(R, dtype=jnp.float32), 16, axis=1)
    S16 = jnp.tile(jnp.eye(16, dtype=jnp.float32), (R, 1))
    cntr = cnt.reshape(2, NP_, R)
    h = _mid_call(acc1.reshape(2, NP_, 128), cntr, o1root, B8, S16)

    acc2 = _sc_pass2(srcp2, sidx2, h)
    out = _fin_call(acc2.reshape(2, NP_, 128), cntr, h, root2,
                    b2.reshape(1, 128), B8, W2.reshape(128, 128))
    return out
